# Initial kernel scaffold; baseline (speedup 1.0000x reference)
import jax
import jax.numpy as jnp
from jax.experimental import pallas as pl


def kernel(x, rel_repr, edge_index, edge_type, edge_norm, time_emd, in_w, loop_w, w_rel, loop_rel, W_e, b_e, W_r, b_r, gamma, beta):
    raise NotImplementedError("write your pallas kernel here")


# speedup vs baseline: 3.5261x; 3.5261x over previous
import functools

import numpy as np
import jax
import jax.numpy as jnp
from jax import lax
from jax.experimental import pallas as pl
from jax.experimental.pallas import tpu as pltpu
from jax.experimental.pallas import tpu_sc as plsc

D = 256
HF = 128
_HIGH = jax.lax.Precision.HIGHEST

_Feye = np.fft.rfft(np.eye(D), axis=-1)
_H_NP = np.concatenate([_Feye.real, _Feye.imag[:, 1:HF]], axis=1).astype(np.float32)


def _unpack_np(p):
    re = p[..., 0:HF + 1]
    z = np.zeros(p.shape[:-1] + (1,))
    im = np.concatenate([z, p[..., HF + 1:D], z], axis=-1)
    return re + 1j * im


_G_NP = np.fft.irfft(_unpack_np(np.eye(D)), n=D, axis=-1).astype(np.float32)
_CIRC_IDX = ((np.arange(D)[:, None] + np.arange(D)[None, :]) % D).astype(np.int32)

NC = 2
NS = 16
NW = NC * NS


def _prep_body(x_ref, pel_ref, fxa_ref, xloop_ref):
    xw = jnp.dot(x_ref[...], pel_ref[...], precision=_HIGH,
                 preferred_element_type=jnp.float32)
    fxa_ref[...] = xw[:, :D]
    xloop_ref[...] = xw[:, D:]


def _rel_body(rel_ref, prw_ref, frb_ref, relout_ref):
    rw = jnp.dot(rel_ref[...], prw_ref[...], precision=_HIGH,
                 preferred_element_type=jnp.float32)
    frb_ref[...] = rw[:, :D]
    relout_ref[...] = rw[:, D:]


GCH = 40


def _gather_body(fxa_hbm, frb_hbm, src_hbm, et_hbm, ga_hbm, gb_hbm,
                 idx_a, idx_b, rows_a, rows_b, sem_a, sem_b, *, e_per_w):
    wid = lax.axis_index("s") * NC + lax.axis_index("c")
    base = wid * e_per_w
    nch = e_per_w // GCH

    def step(g, carry):
        off = base + g * GCH
        pltpu.sync_copy(src_hbm.at[pl.ds(off, GCH)], idx_a)
        pltpu.sync_copy(et_hbm.at[pl.ds(off, GCH)], idx_b)
        pltpu.async_copy(fxa_hbm.at[idx_a], rows_a, sem_a).wait()
        pltpu.async_copy(frb_hbm.at[idx_b], rows_b, sem_b).wait()
        pltpu.sync_copy(rows_a, ga_hbm.at[pl.ds(off, GCH)])
        pltpu.sync_copy(rows_b, gb_hbm.at[pl.ds(off, GCH)])
        return carry

    lax.fori_loop(0, nch, step, 0)


def _edge_body(time_ref, ga_ref, gb_ref, norm_ref, cecr_ref, cst_ref, out_ref):
    tf = jnp.dot(time_ref[...], cecr_ref[...], precision=_HIGH,
                 preferred_element_type=jnp.float32)
    cst = cst_ref[...]
    u = ga_ref[...] + tf[:, :D] + cst[:, :D]
    v = gb_ref[...] + tf[:, D:] + cst[:, D:]
    s = u * v
    ur = jnp.concatenate([u[:, HF:], u[:, :HF]], axis=1)
    t = ur * v
    sl, sh = s[:, :HF], s[:, HF:]
    tl, th = t[:, :HF], t[:, HF:]
    lane = lax.broadcasted_iota(jnp.int32, (1, HF), 1)
    m = lane != 0
    p_lo = sl + jnp.where(m, sh, 0.0)
    p_hi = jnp.where(m, th - tl, sh)
    out_ref[...] = jnp.concatenate([p_lo, p_hi], axis=1) * norm_ref[...]


SCH = 80


def _scatter_body(p_hbm, dst_hbm, zero_hbm, out_hbm,
                  dst_v, rows_v, shared, *, e_per_s, n_rows):
    c = lax.axis_index("c")
    s = lax.axis_index("s")
    rows_per_s = n_rows // NS
    nch = e_per_s // SCH
    pltpu.sync_copy(zero_hbm, shared.at[pl.ds(s * rows_per_s, rows_per_s)])
    plsc.subcore_barrier()

    def step(g, carry):
        e0 = s * e_per_s + g * SCH
        pltpu.sync_copy(dst_hbm.at[pl.ds(e0, SCH)], dst_v)
        pltpu.sync_copy(p_hbm.at[pl.ds(e0, SCH), pl.ds(c * HF, HF)], rows_v)
        pltpu.sync_copy(rows_v, shared.at[dst_v], add=True)
        return carry

    lax.fori_loop(0, nch, step, 0)
    plsc.subcore_barrier()
    pltpu.sync_copy(shared.at[pl.ds(s * rows_per_s, rows_per_s)],
                    out_hbm.at[pl.ds(s * rows_per_s, rows_per_s),
                               pl.ds(c * HF, HF)])


def _post_body(h_ref, xloop_ref, a_ref, pre_ref):
    pre_ref[...] = jnp.dot(h_ref[...], a_ref[...], precision=_HIGH,
                           preferred_element_type=jnp.float32) + xloop_ref[...]


def _bn_body(pre_ref, gamma_ref, beta_ref, out_ref):
    pre = pre_ref[...]
    n = pre.shape[0]
    mean = jnp.sum(pre, axis=0, keepdims=True) / n
    cen = pre - mean
    var = jnp.sum(cen * cen, axis=0, keepdims=True) / n
    out = cen * lax.rsqrt(var + 1e-5) * gamma_ref[...] + beta_ref[...]
    out_ref[...] = jnp.maximum(out, 0.0)


def kernel(x, rel_repr, edge_index, edge_type, edge_norm, time_emd,
           in_w, loop_w, w_rel, loop_rel, W_e, b_e, W_r, b_r, gamma, beta):
    N, _ = x.shape
    R, _ = rel_repr.shape
    E = edge_type.shape[0]
    src = edge_index[0]
    dst = edge_index[1]

    H = jnp.asarray(_H_NP)
    G = jnp.asarray(_G_NP)
    dot = functools.partial(jnp.dot, precision=_HIGH,
                            preferred_element_type=jnp.float32)
    P_e = dot(W_e[:D], H)
    Ce = dot(W_e[D:], H)
    P_r = dot(W_r[:D], H)
    Cr = dot(W_r[D:], H)
    cst = jnp.concatenate([dot(b_e, H), dot(b_r, H)]).reshape(1, 2 * D)
    A = dot(G, in_w) * 0.5
    Lc = dot(loop_rel[0][_CIRC_IDX], loop_w) * 0.5
    PeL = jnp.concatenate([P_e, Lc], axis=1)
    PrW = jnp.concatenate([P_r, w_rel], axis=1)
    CeCr = jnp.concatenate([Ce, Cr], axis=1)

    f32 = jnp.float32
    BN = 2000
    fxa, xloop = pl.pallas_call(
        _prep_body,
        grid=(N // BN,),
        in_specs=[pl.BlockSpec((BN, D), lambda i: (i, 0)),
                  pl.BlockSpec((D, 2 * D), lambda i: (0, 0))],
        out_specs=[pl.BlockSpec((BN, D), lambda i: (i, 0)),
                   pl.BlockSpec((BN, D), lambda i: (i, 0))],
        out_shape=[jax.ShapeDtypeStruct((N, D), f32),
                   jax.ShapeDtypeStruct((N, D), f32)],
    )(x, PeL)
    frb, rel_out = pl.pallas_call(
        _rel_body,
        out_shape=[jax.ShapeDtypeStruct((R, D), f32),
                   jax.ShapeDtypeStruct((R, D), f32)],
    )(rel_repr, PrW)

    e_per_w = E // NW
    mesh = plsc.VectorSubcoreMesh(core_axis_name="c", subcore_axis_name="s")
    gather = functools.partial(
        pl.kernel,
        mesh=mesh,
        out_type=[jax.ShapeDtypeStruct((E, D), f32),
                  jax.ShapeDtypeStruct((E, D), f32)],
        scratch_types=[pltpu.VMEM((GCH,), jnp.int32),
                       pltpu.VMEM((GCH,), jnp.int32),
                       pltpu.VMEM((GCH, D), f32),
                       pltpu.VMEM((GCH, D), f32),
                       pltpu.SemaphoreType.DMA,
                       pltpu.SemaphoreType.DMA],
    )(functools.partial(_gather_body, e_per_w=e_per_w))
    ga, gb = gather(fxa, frb, src, edge_type)

    BE = 2000
    grid = E // BE
    P = pl.pallas_call(
        _edge_body,
        grid=(grid,),
        in_specs=[
            pl.BlockSpec((BE, D), lambda i: (i, 0)),
            pl.BlockSpec((BE, D), lambda i: (i, 0)),
            pl.BlockSpec((BE, D), lambda i: (i, 0)),
            pl.BlockSpec((BE, 1), lambda i: (i, 0)),
            pl.BlockSpec((D, 2 * D), lambda i: (0, 0)),
            pl.BlockSpec((1, 2 * D), lambda i: (0, 0)),
        ],
        out_specs=pl.BlockSpec((BE, D), lambda i: (i, 0)),
        out_shape=jax.ShapeDtypeStruct((E, D), f32),
    )(time_emd, ga, gb, edge_norm.reshape(E, 1), CeCr, cst)

    e_per_s = E // NS
    npad = ((N + 8 * NS - 1) // (8 * NS)) * (8 * NS)
    zero = jnp.zeros((npad // NS, HF), f32)
    scatter = functools.partial(
        pl.kernel,
        mesh=plsc.VectorSubcoreMesh(core_axis_name="c", subcore_axis_name="s"),
        out_type=jax.ShapeDtypeStruct((npad, D), f32),
        scratch_types=[pltpu.VMEM((SCH,), jnp.int32),
                       pltpu.VMEM((SCH, HF), f32),
                       pltpu.VMEM_SHARED((npad, HF), f32)],
    )(functools.partial(_scatter_body, e_per_s=e_per_s, n_rows=npad))
    hsum = scatter(P, dst, zero)[:N]

    pre = pl.pallas_call(
        _post_body,
        grid=(N // BN,),
        in_specs=[pl.BlockSpec((BN, D), lambda i: (i, 0)),
                  pl.BlockSpec((BN, D), lambda i: (i, 0)),
                  pl.BlockSpec((D, D), lambda i: (0, 0))],
        out_specs=pl.BlockSpec((BN, D), lambda i: (i, 0)),
        out_shape=jax.ShapeDtypeStruct((N, D), f32),
    )(hsum, xloop, A)
    BC = 128
    out = pl.pallas_call(
        _bn_body,
        grid=(D // BC,),
        in_specs=[pl.BlockSpec((N, BC), lambda j: (0, j)),
                  pl.BlockSpec((1, BC), lambda j: (0, j)),
                  pl.BlockSpec((1, BC), lambda j: (0, j))],
        out_specs=pl.BlockSpec((N, BC), lambda j: (0, j)),
        out_shape=jax.ShapeDtypeStruct((N, D), f32),
    )(pre, gamma.reshape(1, D), beta.reshape(1, D))

    return out, rel_out


# speedup vs baseline: 4.7480x; 1.3465x over previous
import functools

import numpy as np
import jax
import jax.numpy as jnp
from jax import lax
from jax.experimental import pallas as pl
from jax.experimental.pallas import tpu as pltpu
from jax.experimental.pallas import tpu_sc as plsc

D = 256
HF = 128
_HIGH = jax.lax.Precision.HIGHEST

_Feye = np.fft.rfft(np.eye(D), axis=-1)
_H_NP = np.concatenate([_Feye.real, _Feye.imag[:, 1:HF]], axis=1).astype(np.float32)


def _unpack_np(p):
    re = p[..., 0:HF + 1]
    z = np.zeros(p.shape[:-1] + (1,))
    im = np.concatenate([z, p[..., HF + 1:D], z], axis=-1)
    return re + 1j * im


_G_NP = np.fft.irfft(_unpack_np(np.eye(D)), n=D, axis=-1).astype(np.float32)
_CIRC_IDX = ((np.arange(D)[:, None] + np.arange(D)[None, :]) % D).astype(np.int32)

NC = 2
NS = 16
NW = NC * NS


def _prep_body(x_ref, pel_ref, fxa_ref, xloop_ref):
    xw = jnp.dot(x_ref[...], pel_ref[...], precision=_HIGH,
                 preferred_element_type=jnp.float32)
    fxa_ref[...] = xw[:, :D]
    xloop_ref[...] = xw[:, D:]


def _rel_body(rel_ref, prw_ref, frb_ref, relout_ref):
    rw = jnp.dot(rel_ref[...], prw_ref[...], precision=_HIGH,
                 preferred_element_type=jnp.float32)
    frb_ref[...] = rw[:, :D]
    relout_ref[...] = rw[:, D:]


GCH = 40


def _gather_body(fxa_hbm, frb_hbm, src_hbm, et_hbm, ga_hbm, gb_hbm,
                 src_v, et_v, ra0, ra1, rb0, rb1,
                 gsa0, gsa1, gsb0, gsb1, wsa0, wsa1, wsb0, wsb1,
                 *, e_per_w):
    wid = lax.axis_index("s") * NC + lax.axis_index("c")
    base = wid * e_per_w
    nch = e_per_w // GCH
    pltpu.sync_copy(src_hbm.at[pl.ds(base, e_per_w)], src_v)
    pltpu.sync_copy(et_hbm.at[pl.ds(base, e_per_w)], et_v)
    ra = (ra0, ra1)
    rb = (rb0, rb1)
    gsa = (gsa0, gsa1)
    gsb = (gsb0, gsb1)
    wsa = (wsa0, wsa1)
    wsb = (wsb0, wsb1)

    def issue_gather(c, b):
        sl = pl.ds(c * GCH, GCH)
        pltpu.async_copy(fxa_hbm.at[src_v.at[sl]], ra[b], gsa[b])
        pltpu.async_copy(frb_hbm.at[et_v.at[sl]], rb[b], gsb[b])

    def wait_gather(c, b):
        sl = pl.ds(c * GCH, GCH)
        pltpu.make_async_copy(fxa_hbm.at[src_v.at[sl]], ra[b], gsa[b]).wait()
        pltpu.make_async_copy(frb_hbm.at[et_v.at[sl]], rb[b], gsb[b]).wait()

    def issue_write(c, b):
        sl = pl.ds(base + c * GCH, GCH)
        pltpu.async_copy(ra[b], ga_hbm.at[sl], wsa[b])
        pltpu.async_copy(rb[b], gb_hbm.at[sl], wsb[b])

    def drain_write(b):
        sl = pl.ds(base, GCH)
        pltpu.make_async_copy(ra[b], ga_hbm.at[sl], wsa[b]).wait()
        pltpu.make_async_copy(rb[b], gb_hbm.at[sl], wsb[b]).wait()

    issue_gather(0, 0)

    def pair(g2, carry):
        c = 2 * g2
        wait_gather(c, 0)
        issue_write(c, 0)

        @pl.when(g2 > 0)
        def _():
            drain_write(1)

        issue_gather(c + 1, 1)
        wait_gather(c + 1, 1)
        issue_write(c + 1, 1)
        drain_write(0)

        @pl.when(c + 2 < nch)
        def _():
            issue_gather(c + 2, 0)

        return carry

    lax.fori_loop(0, (nch - 1) // 2, pair, 0)
    wait_gather(nch - 1, 0)
    issue_write(nch - 1, 0)
    drain_write(1)
    drain_write(0)


def _edge_body(time_ref, ga_ref, gb_ref, norm_ref, whi_ref, wlo_ref,
               cst_ref, out_ref):
    tm = time_ref[...]
    th = tm.astype(jnp.bfloat16)
    tl = (tm - th.astype(jnp.float32)).astype(jnp.bfloat16)
    whi = whi_ref[...]
    dot16 = functools.partial(jnp.dot, preferred_element_type=jnp.float32)
    tf = dot16(th, whi) + dot16(th, wlo_ref[...]) + dot16(tl, whi)
    cst = cst_ref[...]
    u = ga_ref[...] + tf[:, :D] + cst[:, :D]
    v = gb_ref[...] + tf[:, D:] + cst[:, D:]
    s = u * v
    ur = jnp.concatenate([u[:, HF:], u[:, :HF]], axis=1)
    t = ur * v
    sl, sh = s[:, :HF], s[:, HF:]
    tl, th = t[:, :HF], t[:, HF:]
    lane = lax.broadcasted_iota(jnp.int32, (1, HF), 1)
    m = lane != 0
    p_lo = sl + jnp.where(m, sh, 0.0)
    p_hi = jnp.where(m, th - tl, sh)
    out_ref[...] = jnp.concatenate([p_lo, p_hi], axis=1) * norm_ref[...]


SCH = 80


def _scatter_body(p_hbm, dst_hbm, zero_hbm, out_hbm,
                  dv0, dv1, rv0, rv1, ds0, ds1, ps0, ps1, shared,
                  *, e_per_s, n_rows):
    c = lax.axis_index("c")
    s = lax.axis_index("s")
    rows_per_s = n_rows // NS
    nch = e_per_s // SCH
    pltpu.sync_copy(zero_hbm, shared.at[pl.ds(s * rows_per_s, rows_per_s)])
    plsc.subcore_barrier()
    dv = (dv0, dv1)
    rv = (rv0, rv1)
    dsem = (ds0, ds1)
    psem = (ps0, ps1)

    def issue_load(g, b):
        e0 = s * e_per_s + g * SCH
        pltpu.async_copy(dst_hbm.at[pl.ds(e0, SCH)], dv[b], dsem[b])
        pltpu.async_copy(p_hbm.at[pl.ds(e0, SCH), pl.ds(c * HF, HF)],
                         rv[b], psem[b])

    def do_chunk(g, b):
        e0 = s * e_per_s + g * SCH
        pltpu.make_async_copy(dst_hbm.at[pl.ds(e0, SCH)], dv[b],
                              dsem[b]).wait()
        pltpu.make_async_copy(p_hbm.at[pl.ds(e0, SCH), pl.ds(c * HF, HF)],
                              rv[b], psem[b]).wait()
        pltpu.sync_copy(rv[b], shared.at[dv[b]], add=True)

    issue_load(0, 0)

    def pair(g2, carry):
        g = 2 * g2
        issue_load(g + 1, 1)
        do_chunk(g, 0)

        @pl.when(g + 2 < nch)
        def _():
            issue_load(g + 2, 0)

        do_chunk(g + 1, 1)
        return carry

    lax.fori_loop(0, (nch - 1) // 2, pair, 0)
    do_chunk(nch - 1, 0)
    plsc.subcore_barrier()
    pltpu.sync_copy(shared.at[pl.ds(s * rows_per_s, rows_per_s)],
                    out_hbm.at[pl.ds(s * rows_per_s, rows_per_s),
                               pl.ds(c * HF, HF)])


def _post_body(h_ref, xloop_ref, a_ref, pre_ref):
    pre_ref[...] = jnp.dot(h_ref[...], a_ref[...], precision=_HIGH,
                           preferred_element_type=jnp.float32) + xloop_ref[...]


def _bn_body(pre_ref, gamma_ref, beta_ref, out_ref):
    pre = pre_ref[...]
    n = pre.shape[0]
    mean = jnp.sum(pre, axis=0, keepdims=True) / n
    cen = pre - mean
    var = jnp.sum(cen * cen, axis=0, keepdims=True) / n
    out = cen * lax.rsqrt(var + 1e-5) * gamma_ref[...] + beta_ref[...]
    out_ref[...] = jnp.maximum(out, 0.0)


def kernel(x, rel_repr, edge_index, edge_type, edge_norm, time_emd,
           in_w, loop_w, w_rel, loop_rel, W_e, b_e, W_r, b_r, gamma, beta):
    N, _ = x.shape
    R, _ = rel_repr.shape
    E = edge_type.shape[0]
    src = edge_index[0]
    dst = edge_index[1]

    H = jnp.asarray(_H_NP)
    G = jnp.asarray(_G_NP)
    dot = functools.partial(jnp.dot, precision=_HIGH,
                            preferred_element_type=jnp.float32)
    P_e = dot(W_e[:D], H)
    Ce = dot(W_e[D:], H)
    P_r = dot(W_r[:D], H)
    Cr = dot(W_r[D:], H)
    cst = jnp.concatenate([dot(b_e, H), dot(b_r, H)]).reshape(1, 2 * D)
    A = dot(G, in_w) * 0.5
    Lc = dot(loop_rel[0][_CIRC_IDX], loop_w) * 0.5
    PeL = jnp.concatenate([P_e, Lc], axis=1)
    PrW = jnp.concatenate([P_r, w_rel], axis=1)
    CeCr = jnp.concatenate([Ce, Cr], axis=1)

    f32 = jnp.float32
    BN = 2000
    fxa, xloop = pl.pallas_call(
        _prep_body,
        grid=(N // BN,),
        in_specs=[pl.BlockSpec((BN, D), lambda i: (i, 0)),
                  pl.BlockSpec((D, 2 * D), lambda i: (0, 0))],
        out_specs=[pl.BlockSpec((BN, D), lambda i: (i, 0)),
                   pl.BlockSpec((BN, D), lambda i: (i, 0))],
        out_shape=[jax.ShapeDtypeStruct((N, D), f32),
                   jax.ShapeDtypeStruct((N, D), f32)],
    )(x, PeL)
    frb, rel_out = pl.pallas_call(
        _rel_body,
        out_shape=[jax.ShapeDtypeStruct((R, D), f32),
                   jax.ShapeDtypeStruct((R, D), f32)],
    )(rel_repr, PrW)

    e_per_w = E // NW
    mesh = plsc.VectorSubcoreMesh(core_axis_name="c", subcore_axis_name="s")
    gather = functools.partial(
        pl.kernel,
        mesh=mesh,
        out_type=[jax.ShapeDtypeStruct((E, D), f32),
                  jax.ShapeDtypeStruct((E, D), f32)],
        scratch_types=[pltpu.VMEM((e_per_w,), jnp.int32),
                       pltpu.VMEM((e_per_w,), jnp.int32)]
                      + [pltpu.VMEM((GCH, D), f32)] * 4
                      + [pltpu.SemaphoreType.DMA] * 8,
    )(functools.partial(_gather_body, e_per_w=e_per_w))
    ga, gb = gather(fxa, frb, src, edge_type)

    BE = 2000
    grid = E // BE
    cecr_hi = CeCr.astype(jnp.bfloat16)
    cecr_lo = (CeCr - cecr_hi.astype(f32)).astype(jnp.bfloat16)
    P = pl.pallas_call(
        _edge_body,
        grid=(grid,),
        in_specs=[
            pl.BlockSpec((BE, D), lambda i: (i, 0)),
            pl.BlockSpec((BE, D), lambda i: (i, 0)),
            pl.BlockSpec((BE, D), lambda i: (i, 0)),
            pl.BlockSpec((BE, 1), lambda i: (i, 0)),
            pl.BlockSpec((D, 2 * D), lambda i: (0, 0)),
            pl.BlockSpec((D, 2 * D), lambda i: (0, 0)),
            pl.BlockSpec((1, 2 * D), lambda i: (0, 0)),
        ],
        out_specs=pl.BlockSpec((BE, D), lambda i: (i, 0)),
        out_shape=jax.ShapeDtypeStruct((E, D), f32),
    )(time_emd, ga, gb, edge_norm.reshape(E, 1), cecr_hi, cecr_lo, cst)

    e_per_s = E // NS
    npad = ((N + 8 * NS - 1) // (8 * NS)) * (8 * NS)
    zero = jnp.zeros((npad // NS, HF), f32)
    scatter = functools.partial(
        pl.kernel,
        mesh=plsc.VectorSubcoreMesh(core_axis_name="c", subcore_axis_name="s"),
        out_type=jax.ShapeDtypeStruct((npad, D), f32),
        scratch_types=[pltpu.VMEM((SCH,), jnp.int32)] * 2
                      + [pltpu.VMEM((SCH, HF), f32)] * 2
                      + [pltpu.SemaphoreType.DMA] * 4
                      + [pltpu.VMEM_SHARED((npad, HF), f32)],
    )(functools.partial(_scatter_body, e_per_s=e_per_s, n_rows=npad))
    hsum = scatter(P, dst, zero)[:N]

    pre = pl.pallas_call(
        _post_body,
        grid=(N // BN,),
        in_specs=[pl.BlockSpec((BN, D), lambda i: (i, 0)),
                  pl.BlockSpec((BN, D), lambda i: (i, 0)),
                  pl.BlockSpec((D, D), lambda i: (0, 0))],
        out_specs=pl.BlockSpec((BN, D), lambda i: (i, 0)),
        out_shape=jax.ShapeDtypeStruct((N, D), f32),
    )(hsum, xloop, A)
    BC = 128
    out = pl.pallas_call(
        _bn_body,
        grid=(D // BC,),
        in_specs=[pl.BlockSpec((N, BC), lambda j: (0, j)),
                  pl.BlockSpec((1, BC), lambda j: (0, j)),
                  pl.BlockSpec((1, BC), lambda j: (0, j))],
        out_specs=pl.BlockSpec((N, BC), lambda j: (0, j)),
        out_shape=jax.ShapeDtypeStruct((N, D), f32),
    )(pre, gamma.reshape(1, D), beta.reshape(1, D))

    return out, rel_out


# speedup vs baseline: 8.0549x; 1.6965x over previous
import functools

import numpy as np
import jax
import jax.numpy as jnp
from jax import lax
from jax.experimental import pallas as pl
from jax.experimental.pallas import tpu as pltpu
from jax.experimental.pallas import tpu_sc as plsc

D = 256
HF = 128
_HIGH = jax.lax.Precision.HIGHEST

_Feye = np.fft.rfft(np.eye(D), axis=-1)
_H_NP = np.concatenate([_Feye.real, _Feye.imag[:, 1:HF]], axis=1).astype(np.float32)


def _unpack_np(p):
    re = p[..., 0:HF + 1]
    z = np.zeros(p.shape[:-1] + (1,))
    im = np.concatenate([z, p[..., HF + 1:D], z], axis=-1)
    return re + 1j * im


_G_NP = np.fft.irfft(_unpack_np(np.eye(D)), n=D, axis=-1).astype(np.float32)
_M_NP = (np.arange(HF) != 0).astype(np.float32)

NC = 2
NS = 16
NW = NC * NS


def _prep_body(x_ref, pel_ref, fxa_ref, xloop_ref):
    xw = jnp.dot(x_ref[...], pel_ref[...], precision=_HIGH,
                 preferred_element_type=jnp.float32)
    fxa_ref[...] = xw[:, :D]
    xloop_ref[...] = xw[:, D:]


def _rel_body(rel_ref, prw_ref, frb_ref, relout_ref):
    rw = jnp.dot(rel_ref[...], prw_ref[...], precision=_HIGH,
                 preferred_element_type=jnp.float32)
    frb_ref[...] = rw[:, :D]
    relout_ref[...] = rw[:, D:]


GCH = 40


def _gather_body(fxa_hbm, frb_hbm, src_hbm, et_hbm, ga_hbm, gb_hbm,
                 src_v, et_v, ra0, ra1, rb0, rb1,
                 gsa0, gsa1, gsb0, gsb1, wsa0, wsa1, wsb0, wsb1,
                 *, e_per_w):
    wid = lax.axis_index("s") * NC + lax.axis_index("c")
    base = wid * e_per_w
    nch = e_per_w // GCH
    pltpu.sync_copy(src_hbm.at[pl.ds(base, e_per_w)], src_v)
    pltpu.sync_copy(et_hbm.at[pl.ds(base, e_per_w)], et_v)
    ra = (ra0, ra1)
    rb = (rb0, rb1)
    gsa = (gsa0, gsa1)
    gsb = (gsb0, gsb1)
    wsa = (wsa0, wsa1)
    wsb = (wsb0, wsb1)

    def issue_gather(c, b):
        sl = pl.ds(c * GCH, GCH)
        pltpu.async_copy(fxa_hbm.at[src_v.at[sl]], ra[b], gsa[b])
        pltpu.async_copy(frb_hbm.at[et_v.at[sl]], rb[b], gsb[b])

    def wait_gather(c, b):
        sl = pl.ds(c * GCH, GCH)
        pltpu.make_async_copy(fxa_hbm.at[src_v.at[sl]], ra[b], gsa[b]).wait()
        pltpu.make_async_copy(frb_hbm.at[et_v.at[sl]], rb[b], gsb[b]).wait()

    def issue_write(c, b):
        sl = pl.ds(base + c * GCH, GCH)
        pltpu.async_copy(ra[b], ga_hbm.at[sl], wsa[b])
        pltpu.async_copy(rb[b], gb_hbm.at[sl], wsb[b])

    def drain_write(b):
        sl = pl.ds(base, GCH)
        pltpu.make_async_copy(ra[b], ga_hbm.at[sl], wsa[b]).wait()
        pltpu.make_async_copy(rb[b], gb_hbm.at[sl], wsb[b]).wait()

    issue_gather(0, 0)

    def pair(g2, carry):
        c = 2 * g2
        wait_gather(c, 0)
        issue_write(c, 0)

        @pl.when(g2 > 0)
        def _():
            drain_write(1)

        issue_gather(c + 1, 1)
        wait_gather(c + 1, 1)
        issue_write(c + 1, 1)
        drain_write(0)

        @pl.when(c + 2 < nch)
        def _():
            issue_gather(c + 2, 0)

        return carry

    lax.fori_loop(0, (nch - 1) // 2, pair, 0)
    wait_gather(nch - 1, 0)
    issue_write(nch - 1, 0)
    drain_write(1)
    drain_write(0)


def _edge_body(time_ref, ga_ref, gb_ref, norm_ref, whi_ref, wlo_ref,
               cst_ref, out_ref):
    tm = time_ref[...]
    th = tm.astype(jnp.bfloat16)
    tl = (tm - th.astype(jnp.float32)).astype(jnp.bfloat16)
    whi = whi_ref[...]
    dot16 = functools.partial(jnp.dot, preferred_element_type=jnp.float32)
    tf = dot16(th, whi) + dot16(th, wlo_ref[...]) + dot16(tl, whi)
    cst = cst_ref[...]
    u = ga_ref[...] + tf[:, :D] + cst[:, :D]
    v = gb_ref[...] + tf[:, D:] + cst[:, D:]
    s = u * v
    ur = jnp.concatenate([u[:, HF:], u[:, :HF]], axis=1)
    t = ur * v
    sl, sh = s[:, :HF], s[:, HF:]
    tl, th = t[:, :HF], t[:, HF:]
    lane = lax.broadcasted_iota(jnp.int32, (1, HF), 1)
    m = lane != 0
    p_lo = sl + jnp.where(m, sh, 0.0)
    p_hi = jnp.where(m, th - tl, sh)
    out_ref[...] = jnp.concatenate([p_lo, p_hi], axis=1) * norm_ref[...]


SCH = 80


def _scatter_body(p_hbm, dst_hbm, zero_hbm, out_hbm,
                  dv0, dv1, rv0, rv1, ds0, ds1, ps0, ps1, shared,
                  *, e_per_s, n_rows):
    c = lax.axis_index("c")
    s = lax.axis_index("s")
    rows_per_s = n_rows // NS
    nch = e_per_s // SCH
    pltpu.sync_copy(zero_hbm, shared.at[pl.ds(s * rows_per_s, rows_per_s)])
    plsc.subcore_barrier()
    dv = (dv0, dv1)
    rv = (rv0, rv1)
    dsem = (ds0, ds1)
    psem = (ps0, ps1)

    def issue_load(g, b):
        e0 = s * e_per_s + g * SCH
        pltpu.async_copy(dst_hbm.at[pl.ds(e0, SCH)], dv[b], dsem[b])
        pltpu.async_copy(p_hbm.at[pl.ds(e0, SCH), pl.ds(c * HF, HF)],
                         rv[b], psem[b])

    def do_chunk(g, b):
        e0 = s * e_per_s + g * SCH
        pltpu.make_async_copy(dst_hbm.at[pl.ds(e0, SCH)], dv[b],
                              dsem[b]).wait()
        pltpu.make_async_copy(p_hbm.at[pl.ds(e0, SCH), pl.ds(c * HF, HF)],
                              rv[b], psem[b]).wait()
        pltpu.sync_copy(rv[b], shared.at[dv[b]], add=True)

    issue_load(0, 0)

    def pair(g2, carry):
        g = 2 * g2
        issue_load(g + 1, 1)
        do_chunk(g, 0)

        @pl.when(g + 2 < nch)
        def _():
            issue_load(g + 2, 0)

        do_chunk(g + 1, 1)
        return carry

    lax.fori_loop(0, (nch - 1) // 2, pair, 0)
    do_chunk(nch - 1, 0)
    plsc.subcore_barrier()
    pltpu.sync_copy(shared.at[pl.ds(s * rows_per_s, rows_per_s)],
                    out_hbm.at[pl.ds(s * rows_per_s, rows_per_s),
                               pl.ds(c * HF, HF)])


def _post_body(h_ref, xloop_ref, a_ref, pre_ref):
    pre_ref[...] = jnp.dot(h_ref[...], a_ref[...], precision=_HIGH,
                           preferred_element_type=jnp.float32) + xloop_ref[...]


def _bn_body(pre_ref, gamma_ref, beta_ref, out_ref):
    pre = pre_ref[...]
    n = pre.shape[0]
    mean = jnp.sum(pre, axis=0, keepdims=True) / n
    cen = pre - mean
    var = jnp.sum(cen * cen, axis=0, keepdims=True) / n
    out = cen * lax.rsqrt(var + 1e-5) * gamma_ref[...] + beta_ref[...]
    out_ref[...] = jnp.maximum(out, 0.0)


def kernel(x, rel_repr, edge_index, edge_type, edge_norm, time_emd,
           in_w, loop_w, w_rel, loop_rel, W_e, b_e, W_r, b_r, gamma, beta):
    N, _ = x.shape
    R, _ = rel_repr.shape
    E = edge_type.shape[0]
    src = edge_index[0]
    dst = edge_index[1]

    H = jnp.asarray(_H_NP)
    G = jnp.asarray(_G_NP)
    dot = functools.partial(jnp.dot, precision=_HIGH,
                            preferred_element_type=jnp.float32)
    P_e = dot(W_e[:D], H)
    Ce = dot(W_e[D:], H)
    P_r = dot(W_r[:D], H)
    Cr = dot(W_r[D:], H)
    cst = jnp.concatenate([dot(b_e, H), dot(b_r, H)]).reshape(1, 2 * D)
    A = dot(G, in_w) * 0.5
    v = dot(loop_rel, H)[0]
    va, vb = v[:HF], v[HF:]
    m = jnp.asarray(_M_NP)
    Dv = jnp.concatenate(
        [jnp.concatenate([jnp.diag(va), jnp.diag(m * vb)], axis=1),
         jnp.concatenate([jnp.diag(m * vb), jnp.diag((1 - m) * vb - m * va)],
                         axis=1)], axis=0)
    Lc = dot(H, dot(Dv, dot(G, loop_w))) * 0.5
    PeL = jnp.concatenate([P_e, Lc], axis=1)
    PrW = jnp.concatenate([P_r, w_rel], axis=1)
    CeCr = jnp.concatenate([Ce, Cr], axis=1)

    f32 = jnp.float32
    BN = 2000
    fxa, xloop = pl.pallas_call(
        _prep_body,
        grid=(N // BN,),
        in_specs=[pl.BlockSpec((BN, D), lambda i: (i, 0)),
                  pl.BlockSpec((D, 2 * D), lambda i: (0, 0))],
        out_specs=[pl.BlockSpec((BN, D), lambda i: (i, 0)),
                   pl.BlockSpec((BN, D), lambda i: (i, 0))],
        out_shape=[jax.ShapeDtypeStruct((N, D), f32),
                   jax.ShapeDtypeStruct((N, D), f32)],
    )(x, PeL)
    frb, rel_out = pl.pallas_call(
        _rel_body,
        out_shape=[jax.ShapeDtypeStruct((R, D), f32),
                   jax.ShapeDtypeStruct((R, D), f32)],
    )(rel_repr, PrW)

    e_per_w = E // NW
    mesh = plsc.VectorSubcoreMesh(core_axis_name="c", subcore_axis_name="s")
    gather = functools.partial(
        pl.kernel,
        mesh=mesh,
        out_type=[jax.ShapeDtypeStruct((E, D), f32),
                  jax.ShapeDtypeStruct((E, D), f32)],
        scratch_types=[pltpu.VMEM((e_per_w,), jnp.int32),
                       pltpu.VMEM((e_per_w,), jnp.int32)]
                      + [pltpu.VMEM((GCH, D), f32)] * 4
                      + [pltpu.SemaphoreType.DMA] * 8,
    )(functools.partial(_gather_body, e_per_w=e_per_w))
    ga, gb = gather(fxa, frb, src, edge_type)

    BE = 2000
    grid = E // BE
    cecr_hi = CeCr.astype(jnp.bfloat16)
    cecr_lo = (CeCr - cecr_hi.astype(f32)).astype(jnp.bfloat16)
    P = pl.pallas_call(
        _edge_body,
        grid=(grid,),
        in_specs=[
            pl.BlockSpec((BE, D), lambda i: (i, 0)),
            pl.BlockSpec((BE, D), lambda i: (i, 0)),
            pl.BlockSpec((BE, D), lambda i: (i, 0)),
            pl.BlockSpec((BE, 1), lambda i: (i, 0)),
            pl.BlockSpec((D, 2 * D), lambda i: (0, 0)),
            pl.BlockSpec((D, 2 * D), lambda i: (0, 0)),
            pl.BlockSpec((1, 2 * D), lambda i: (0, 0)),
        ],
        out_specs=pl.BlockSpec((BE, D), lambda i: (i, 0)),
        out_shape=jax.ShapeDtypeStruct((E, D), f32),
    )(time_emd, ga, gb, edge_norm.reshape(E, 1), cecr_hi, cecr_lo, cst)

    e_per_s = E // NS
    npad = ((N + 8 * NS - 1) // (8 * NS)) * (8 * NS)
    zero = jnp.zeros((npad // NS, HF), f32)
    scatter = functools.partial(
        pl.kernel,
        mesh=plsc.VectorSubcoreMesh(core_axis_name="c", subcore_axis_name="s"),
        out_type=jax.ShapeDtypeStruct((npad, D), f32),
        scratch_types=[pltpu.VMEM((SCH,), jnp.int32)] * 2
                      + [pltpu.VMEM((SCH, HF), f32)] * 2
                      + [pltpu.SemaphoreType.DMA] * 4
                      + [pltpu.VMEM_SHARED((npad, HF), f32)],
    )(functools.partial(_scatter_body, e_per_s=e_per_s, n_rows=npad))
    hsum = scatter(P, dst, zero)[:N]

    pre = pl.pallas_call(
        _post_body,
        grid=(N // BN,),
        in_specs=[pl.BlockSpec((BN, D), lambda i: (i, 0)),
                  pl.BlockSpec((BN, D), lambda i: (i, 0)),
                  pl.BlockSpec((D, D), lambda i: (0, 0))],
        out_specs=pl.BlockSpec((BN, D), lambda i: (i, 0)),
        out_shape=jax.ShapeDtypeStruct((N, D), f32),
    )(hsum, xloop, A)
    BC = 128
    out = pl.pallas_call(
        _bn_body,
        grid=(D // BC,),
        in_specs=[pl.BlockSpec((N, BC), lambda j: (0, j)),
                  pl.BlockSpec((1, BC), lambda j: (0, j)),
                  pl.BlockSpec((1, BC), lambda j: (0, j))],
        out_specs=pl.BlockSpec((N, BC), lambda j: (0, j)),
        out_shape=jax.ShapeDtypeStruct((N, D), f32),
    )(pre, gamma.reshape(1, D), beta.reshape(1, D))

    return out, rel_out


# speedup vs baseline: 8.3510x; 1.0368x over previous
import functools

import numpy as np
import jax
import jax.numpy as jnp
from jax import lax
from jax.experimental import pallas as pl
from jax.experimental.pallas import tpu as pltpu
from jax.experimental.pallas import tpu_sc as plsc

D = 256
HF = 128
_HIGH = jax.lax.Precision.HIGHEST

_Feye = np.fft.rfft(np.eye(D), axis=-1)
_H_NP = np.concatenate([_Feye.real, _Feye.imag[:, 1:HF]], axis=1).astype(np.float32)


def _unpack_np(p):
    re = p[..., 0:HF + 1]
    z = np.zeros(p.shape[:-1] + (1,))
    im = np.concatenate([z, p[..., HF + 1:D], z], axis=-1)
    return re + 1j * im


_G_NP = np.fft.irfft(_unpack_np(np.eye(D)), n=D, axis=-1).astype(np.float32)
_M_NP = (np.arange(HF) != 0).astype(np.float32)

NC = 2
NS = 16
NW = NC * NS


def _prep_body(x_ref, pel_ref, fxa_ref, xloop_ref):
    xw = jnp.dot(x_ref[...], pel_ref[...], precision=_HIGH,
                 preferred_element_type=jnp.float32)
    fxa_ref[...] = xw[:, :D]
    xloop_ref[...] = xw[:, D:]


def _rel_body(rel_ref, prw_ref, frb_ref, relout_ref):
    rw = jnp.dot(rel_ref[...], prw_ref[...], precision=_HIGH,
                 preferred_element_type=jnp.float32)
    frb_ref[...] = rw[:, :D]
    relout_ref[...] = rw[:, D:]


GCH = 40


def _gather_body(fxa_hbm, frb_hbm, src_hbm, et_hbm, ga_hbm, gb_hbm,
                 src_v, et_v, ra0, ra1, rb0, rb1,
                 gsa0, gsa1, gsb0, gsb1, wsa0, wsa1, wsb0, wsb1,
                 *, e_per_w):
    wid = lax.axis_index("s") * NC + lax.axis_index("c")
    base = wid * e_per_w
    nch = e_per_w // GCH
    pltpu.sync_copy(src_hbm.at[pl.ds(base, e_per_w)], src_v)
    pltpu.sync_copy(et_hbm.at[pl.ds(base, e_per_w)], et_v)
    ra = (ra0, ra1)
    rb = (rb0, rb1)
    gsa = (gsa0, gsa1)
    gsb = (gsb0, gsb1)
    wsa = (wsa0, wsa1)
    wsb = (wsb0, wsb1)

    def issue_gather(c, b):
        sl = pl.ds(c * GCH, GCH)
        pltpu.async_copy(fxa_hbm.at[src_v.at[sl]], ra[b], gsa[b])
        pltpu.async_copy(frb_hbm.at[et_v.at[sl]], rb[b], gsb[b])

    def wait_gather(c, b):
        sl = pl.ds(c * GCH, GCH)
        pltpu.make_async_copy(fxa_hbm.at[src_v.at[sl]], ra[b], gsa[b]).wait()
        pltpu.make_async_copy(frb_hbm.at[et_v.at[sl]], rb[b], gsb[b]).wait()

    def issue_write(c, b):
        sl = pl.ds(base + c * GCH, GCH)
        pltpu.async_copy(ra[b], ga_hbm.at[sl], wsa[b])
        pltpu.async_copy(rb[b], gb_hbm.at[sl], wsb[b])

    def drain_write(b):
        sl = pl.ds(base, GCH)
        pltpu.make_async_copy(ra[b], ga_hbm.at[sl], wsa[b]).wait()
        pltpu.make_async_copy(rb[b], gb_hbm.at[sl], wsb[b]).wait()

    issue_gather(0, 0)

    def pair(g2, carry):
        c = 2 * g2
        wait_gather(c, 0)
        issue_write(c, 0)

        @pl.when(g2 > 0)
        def _():
            drain_write(1)

        issue_gather(c + 1, 1)
        wait_gather(c + 1, 1)
        issue_write(c + 1, 1)
        drain_write(0)

        @pl.when(c + 2 < nch)
        def _():
            issue_gather(c + 2, 0)

        return carry

    lax.fori_loop(0, (nch - 1) // 2, pair, 0)
    wait_gather(nch - 1, 0)
    issue_write(nch - 1, 0)
    drain_write(1)
    drain_write(0)


def _edge_body(time_ref, ga_ref, gb_ref, norm_ref, whi_ref, wlo_ref,
               cst_ref, out_ref):
    th = time_ref[...].astype(jnp.bfloat16)
    dot16 = functools.partial(jnp.dot, preferred_element_type=jnp.float32)
    tf = dot16(th, whi_ref[...]) + dot16(th, wlo_ref[...])
    cst = cst_ref[...]
    u = ga_ref[...] + tf[:, :D] + cst[:, :D]
    v = gb_ref[...] + tf[:, D:] + cst[:, D:]
    s = u * v
    ur = jnp.concatenate([u[:, HF:], u[:, :HF]], axis=1)
    t = ur * v
    sl, sh = s[:, :HF], s[:, HF:]
    tl, th = t[:, :HF], t[:, HF:]
    lane = lax.broadcasted_iota(jnp.int32, (1, HF), 1)
    m = lane != 0
    p_lo = sl + jnp.where(m, sh, 0.0)
    p_hi = jnp.where(m, th - tl, sh)
    out_ref[...] = jnp.concatenate([p_lo, p_hi], axis=1) * norm_ref[...]


SCH = 80


def _scatter_body(p_hbm, dst_hbm, zero_hbm, out_hbm,
                  dv0, dv1, rv0, rv1, ds0, ds1, ps0, ps1, shared,
                  *, e_per_s, n_rows):
    c = lax.axis_index("c")
    s = lax.axis_index("s")
    rows_per_s = n_rows // NS
    nch = e_per_s // SCH
    pltpu.sync_copy(zero_hbm, shared.at[pl.ds(s * rows_per_s, rows_per_s)])
    plsc.subcore_barrier()
    dv = (dv0, dv1)
    rv = (rv0, rv1)
    dsem = (ds0, ds1)
    psem = (ps0, ps1)

    def issue_load(g, b):
        e0 = s * e_per_s + g * SCH
        pltpu.async_copy(dst_hbm.at[pl.ds(e0, SCH)], dv[b], dsem[b])
        pltpu.async_copy(p_hbm.at[pl.ds(e0, SCH), pl.ds(c * HF, HF)],
                         rv[b], psem[b])

    def do_chunk(g, b):
        e0 = s * e_per_s + g * SCH
        pltpu.make_async_copy(dst_hbm.at[pl.ds(e0, SCH)], dv[b],
                              dsem[b]).wait()
        pltpu.make_async_copy(p_hbm.at[pl.ds(e0, SCH), pl.ds(c * HF, HF)],
                              rv[b], psem[b]).wait()
        pltpu.sync_copy(rv[b], shared.at[dv[b]], add=True)

    issue_load(0, 0)

    def pair(g2, carry):
        g = 2 * g2
        issue_load(g + 1, 1)
        do_chunk(g, 0)

        @pl.when(g + 2 < nch)
        def _():
            issue_load(g + 2, 0)

        do_chunk(g + 1, 1)
        return carry

    lax.fori_loop(0, (nch - 1) // 2, pair, 0)
    do_chunk(nch - 1, 0)
    plsc.subcore_barrier()
    pltpu.sync_copy(shared.at[pl.ds(s * rows_per_s, rows_per_s)],
                    out_hbm.at[pl.ds(s * rows_per_s, rows_per_s),
                               pl.ds(c * HF, HF)])


def _post_body(h_ref, xloop_ref, a_ref, pre_ref):
    pre_ref[...] = jnp.dot(h_ref[...], a_ref[...], precision=_HIGH,
                           preferred_element_type=jnp.float32) + xloop_ref[...]


def _bn_body(pre_ref, gamma_ref, beta_ref, out_ref):
    pre = pre_ref[...]
    n = pre.shape[0]
    mean = jnp.sum(pre, axis=0, keepdims=True) / n
    cen = pre - mean
    var = jnp.sum(cen * cen, axis=0, keepdims=True) / n
    out = cen * lax.rsqrt(var + 1e-5) * gamma_ref[...] + beta_ref[...]
    out_ref[...] = jnp.maximum(out, 0.0)


def kernel(x, rel_repr, edge_index, edge_type, edge_norm, time_emd,
           in_w, loop_w, w_rel, loop_rel, W_e, b_e, W_r, b_r, gamma, beta):
    N, _ = x.shape
    R, _ = rel_repr.shape
    E = edge_type.shape[0]
    src = edge_index[0]
    dst = edge_index[1]

    H = jnp.asarray(_H_NP)
    G = jnp.asarray(_G_NP)
    dot = functools.partial(jnp.dot, precision=_HIGH,
                            preferred_element_type=jnp.float32)
    P_e = dot(W_e[:D], H)
    Ce = dot(W_e[D:], H)
    P_r = dot(W_r[:D], H)
    Cr = dot(W_r[D:], H)
    cst = jnp.concatenate([dot(b_e, H), dot(b_r, H)]).reshape(1, 2 * D)
    A = dot(G, in_w) * 0.5
    v = dot(loop_rel, H)[0]
    va, vb = v[:HF], v[HF:]
    m = jnp.asarray(_M_NP)
    Dv = jnp.concatenate(
        [jnp.concatenate([jnp.diag(va), jnp.diag(m * vb)], axis=1),
         jnp.concatenate([jnp.diag(m * vb), jnp.diag((1 - m) * vb - m * va)],
                         axis=1)], axis=0)
    Lc = dot(H, dot(Dv, dot(G, loop_w))) * 0.5
    PeL = jnp.concatenate([P_e, Lc], axis=1)
    PrW = jnp.concatenate([P_r, w_rel], axis=1)
    CeCr = jnp.concatenate([Ce, Cr], axis=1)

    f32 = jnp.float32
    BN = 2000
    fxa, xloop = pl.pallas_call(
        _prep_body,
        grid=(N // BN,),
        in_specs=[pl.BlockSpec((BN, D), lambda i: (i, 0)),
                  pl.BlockSpec((D, 2 * D), lambda i: (0, 0))],
        out_specs=[pl.BlockSpec((BN, D), lambda i: (i, 0)),
                   pl.BlockSpec((BN, D), lambda i: (i, 0))],
        out_shape=[jax.ShapeDtypeStruct((N, D), f32),
                   jax.ShapeDtypeStruct((N, D), f32)],
    )(x, PeL)
    frb, rel_out = pl.pallas_call(
        _rel_body,
        out_shape=[jax.ShapeDtypeStruct((R, D), f32),
                   jax.ShapeDtypeStruct((R, D), f32)],
    )(rel_repr, PrW)

    e_per_w = E // NW
    mesh = plsc.VectorSubcoreMesh(core_axis_name="c", subcore_axis_name="s")
    gather = functools.partial(
        pl.kernel,
        mesh=mesh,
        out_type=[jax.ShapeDtypeStruct((E, D), f32),
                  jax.ShapeDtypeStruct((E, D), f32)],
        scratch_types=[pltpu.VMEM((e_per_w,), jnp.int32),
                       pltpu.VMEM((e_per_w,), jnp.int32)]
                      + [pltpu.VMEM((GCH, D), f32)] * 4
                      + [pltpu.SemaphoreType.DMA] * 8,
    )(functools.partial(_gather_body, e_per_w=e_per_w))
    ga, gb = gather(fxa, frb, src, edge_type)

    BE = 2000
    grid = E // BE
    cecr_hi = CeCr.astype(jnp.bfloat16)
    cecr_lo = (CeCr - cecr_hi.astype(f32)).astype(jnp.bfloat16)
    P = pl.pallas_call(
        _edge_body,
        grid=(grid,),
        in_specs=[
            pl.BlockSpec((BE, D), lambda i: (i, 0)),
            pl.BlockSpec((BE, D), lambda i: (i, 0)),
            pl.BlockSpec((BE, D), lambda i: (i, 0)),
            pl.BlockSpec((BE, 1), lambda i: (i, 0)),
            pl.BlockSpec((D, 2 * D), lambda i: (0, 0)),
            pl.BlockSpec((D, 2 * D), lambda i: (0, 0)),
            pl.BlockSpec((1, 2 * D), lambda i: (0, 0)),
        ],
        out_specs=pl.BlockSpec((BE, D), lambda i: (i, 0)),
        out_shape=jax.ShapeDtypeStruct((E, D), f32),
    )(time_emd, ga, gb, edge_norm.reshape(E, 1), cecr_hi, cecr_lo, cst)

    e_per_s = E // NS
    npad = ((N + 8 * NS - 1) // (8 * NS)) * (8 * NS)
    zero = jnp.zeros((npad // NS, HF), f32)
    scatter = functools.partial(
        pl.kernel,
        mesh=plsc.VectorSubcoreMesh(core_axis_name="c", subcore_axis_name="s"),
        out_type=jax.ShapeDtypeStruct((npad, D), f32),
        scratch_types=[pltpu.VMEM((SCH,), jnp.int32)] * 2
                      + [pltpu.VMEM((SCH, HF), f32)] * 2
                      + [pltpu.SemaphoreType.DMA] * 4
                      + [pltpu.VMEM_SHARED((npad, HF), f32)],
    )(functools.partial(_scatter_body, e_per_s=e_per_s, n_rows=npad))
    hsum = scatter(P, dst, zero)

    pre = pl.pallas_call(
        _post_body,
        grid=(N // BN,),
        in_specs=[pl.BlockSpec((BN, D), lambda i: (i, 0)),
                  pl.BlockSpec((BN, D), lambda i: (i, 0)),
                  pl.BlockSpec((D, D), lambda i: (0, 0))],
        out_specs=pl.BlockSpec((BN, D), lambda i: (i, 0)),
        out_shape=jax.ShapeDtypeStruct((N, D), f32),
    )(hsum, xloop, A)
    BC = 128
    out = pl.pallas_call(
        _bn_body,
        grid=(D // BC,),
        in_specs=[pl.BlockSpec((N, BC), lambda j: (0, j)),
                  pl.BlockSpec((1, BC), lambda j: (0, j)),
                  pl.BlockSpec((1, BC), lambda j: (0, j))],
        out_specs=pl.BlockSpec((N, BC), lambda j: (0, j)),
        out_shape=jax.ShapeDtypeStruct((N, D), f32),
    )(pre, gamma.reshape(1, D), beta.reshape(1, D))

    return out, rel_out


# speedup vs baseline: 8.6571x; 1.0367x over previous
import functools

import numpy as np
import jax
import jax.numpy as jnp
from jax import lax
from jax.experimental import pallas as pl
from jax.experimental.pallas import tpu as pltpu
from jax.experimental.pallas import tpu_sc as plsc

D = 256
HF = 128
_HIGH = jax.lax.Precision.HIGHEST

_Feye = np.fft.rfft(np.eye(D), axis=-1)
_H_NP = np.concatenate([_Feye.real, _Feye.imag[:, 1:HF]], axis=1).astype(np.float32)


def _unpack_np(p):
    re = p[..., 0:HF + 1]
    z = np.zeros(p.shape[:-1] + (1,))
    im = np.concatenate([z, p[..., HF + 1:D], z], axis=-1)
    return re + 1j * im


_G_NP = np.fft.irfft(_unpack_np(np.eye(D)), n=D, axis=-1).astype(np.float32)
_M_NP = (np.arange(HF) != 0).astype(np.float32)

NC = 2
NS = 16
NW = NC * NS


def _prep_body(x_ref, pel_ref, fxa_ref, xloop_ref):
    xw = jnp.dot(x_ref[...], pel_ref[...], precision=_HIGH,
                 preferred_element_type=jnp.float32)
    fxa_ref[...] = xw[:, :D]
    xloop_ref[...] = xw[:, D:]


def _rel_body(rel_ref, prw_ref, frb_ref, relout_ref):
    rw = jnp.dot(rel_ref[...], prw_ref[...], precision=_HIGH,
                 preferred_element_type=jnp.float32)
    frb_ref[...] = rw[:, :D]
    relout_ref[...] = rw[:, D:]


GCH = 40


NB = 5


def _gather_body(fxa_hbm, frb_hbm, src_hbm, et_hbm, ga_hbm, gb_hbm,
                 src_v, et_v, *bufs, e_per_w):
    ra = bufs[0:NB]
    rb = bufs[NB:2 * NB]
    gsa = bufs[2 * NB:3 * NB]
    gsb = bufs[3 * NB:4 * NB]
    wsa = bufs[4 * NB:5 * NB]
    wsb = bufs[5 * NB:6 * NB]
    wid = lax.axis_index("s") * NC + lax.axis_index("c")
    base = wid * e_per_w
    nch = e_per_w // GCH
    pltpu.sync_copy(src_hbm.at[pl.ds(base, e_per_w)], src_v)
    pltpu.sync_copy(et_hbm.at[pl.ds(base, e_per_w)], et_v)

    def issue_gather(c, b):
        sl = pl.ds(c * GCH, GCH)
        pltpu.async_copy(fxa_hbm.at[src_v.at[sl]], ra[b], gsa[b])
        pltpu.async_copy(frb_hbm.at[et_v.at[sl]], rb[b], gsb[b])

    def wait_gather(c, b):
        sl = pl.ds(c * GCH, GCH)
        pltpu.make_async_copy(fxa_hbm.at[src_v.at[sl]], ra[b], gsa[b]).wait()
        pltpu.make_async_copy(frb_hbm.at[et_v.at[sl]], rb[b], gsb[b]).wait()

    def issue_write(c, b):
        sl = pl.ds(base + c * GCH, GCH)
        pltpu.async_copy(ra[b], ga_hbm.at[sl], wsa[b])
        pltpu.async_copy(rb[b], gb_hbm.at[sl], wsb[b])

    def drain_write(b):
        sl = pl.ds(base, GCH)
        pltpu.make_async_copy(ra[b], ga_hbm.at[sl], wsa[b]).wait()
        pltpu.make_async_copy(rb[b], gb_hbm.at[sl], wsb[b]).wait()

    for b in range(NB):
        issue_gather(b, b)

    def group(g, carry):
        c0 = g * NB
        for b in range(NB):
            wait_gather(c0 + b, b)
            issue_write(c0 + b, b)
        for b in range(NB):
            drain_write(b)

            @pl.when(c0 + NB + b < nch)
            def _():
                issue_gather(c0 + NB + b, b)

        return carry

    lax.fori_loop(0, nch // NB, group, 0)


def _edge_body(time_ref, ga_ref, gb_ref, norm_ref, whi_ref, wlo_ref,
               cst_ref, out_ref):
    th = time_ref[...].astype(jnp.bfloat16)
    dot16 = functools.partial(jnp.dot, preferred_element_type=jnp.float32)
    tf = dot16(th, whi_ref[...]) + dot16(th, wlo_ref[...])
    cst = cst_ref[...]
    u = ga_ref[...] + tf[:, :D] + cst[:, :D]
    v = gb_ref[...] + tf[:, D:] + cst[:, D:]
    s = u * v
    ur = jnp.concatenate([u[:, HF:], u[:, :HF]], axis=1)
    t = ur * v
    sl, sh = s[:, :HF], s[:, HF:]
    tl, th = t[:, :HF], t[:, HF:]
    lane = lax.broadcasted_iota(jnp.int32, (1, HF), 1)
    m = lane != 0
    p_lo = sl + jnp.where(m, sh, 0.0)
    p_hi = jnp.where(m, th - tl, sh)
    out_ref[...] = jnp.concatenate([p_lo, p_hi], axis=1) * norm_ref[...]


SCH = 40


def _scatter_body(p_hbm, dst_hbm, zero_hbm, out_hbm, *bufs,
                  e_per_s, n_rows):
    dv = bufs[0:NB]
    rv = bufs[NB:2 * NB]
    dsem = bufs[2 * NB:3 * NB]
    psem = bufs[3 * NB:4 * NB]
    shared = bufs[4 * NB]
    c = lax.axis_index("c")
    s = lax.axis_index("s")
    rows_per_s = n_rows // NS
    nch = e_per_s // SCH
    pltpu.sync_copy(zero_hbm, shared.at[pl.ds(s * rows_per_s, rows_per_s)])
    plsc.subcore_barrier()

    def issue_load(g, b):
        e0 = s * e_per_s + g * SCH
        pltpu.async_copy(dst_hbm.at[pl.ds(e0, SCH)], dv[b], dsem[b])
        pltpu.async_copy(p_hbm.at[pl.ds(e0, SCH), pl.ds(c * HF, HF)],
                         rv[b], psem[b])

    def do_chunk(g, b):
        e0 = s * e_per_s + g * SCH
        pltpu.make_async_copy(dst_hbm.at[pl.ds(e0, SCH)], dv[b],
                              dsem[b]).wait()
        pltpu.make_async_copy(p_hbm.at[pl.ds(e0, SCH), pl.ds(c * HF, HF)],
                              rv[b], psem[b]).wait()
        pltpu.sync_copy(rv[b], shared.at[dv[b]], add=True)

    for b in range(NB):
        issue_load(b, b)

    def group(g, carry):
        c0 = g * NB
        for b in range(NB):
            do_chunk(c0 + b, b)

            @pl.when(c0 + NB + b < nch)
            def _():
                issue_load(c0 + NB + b, b)

        return carry

    lax.fori_loop(0, nch // NB, group, 0)
    plsc.subcore_barrier()
    pltpu.sync_copy(shared.at[pl.ds(s * rows_per_s, rows_per_s)],
                    out_hbm.at[pl.ds(s * rows_per_s, rows_per_s),
                               pl.ds(c * HF, HF)])


def _post_body(h_ref, xloop_ref, a_ref, pre_ref):
    pre_ref[...] = jnp.dot(h_ref[...], a_ref[...], precision=_HIGH,
                           preferred_element_type=jnp.float32) + xloop_ref[...]


def _bn_body(pre_ref, gamma_ref, beta_ref, out_ref):
    pre = pre_ref[...]
    n = pre.shape[0]
    mean = jnp.sum(pre, axis=0, keepdims=True) / n
    cen = pre - mean
    var = jnp.sum(cen * cen, axis=0, keepdims=True) / n
    out = cen * lax.rsqrt(var + 1e-5) * gamma_ref[...] + beta_ref[...]
    out_ref[...] = jnp.maximum(out, 0.0)


def kernel(x, rel_repr, edge_index, edge_type, edge_norm, time_emd,
           in_w, loop_w, w_rel, loop_rel, W_e, b_e, W_r, b_r, gamma, beta):
    N, _ = x.shape
    R, _ = rel_repr.shape
    E = edge_type.shape[0]
    src = edge_index[0]
    dst = edge_index[1]

    H = jnp.asarray(_H_NP)
    G = jnp.asarray(_G_NP)
    dot = functools.partial(jnp.dot, precision=_HIGH,
                            preferred_element_type=jnp.float32)
    P_e = dot(W_e[:D], H)
    Ce = dot(W_e[D:], H)
    P_r = dot(W_r[:D], H)
    Cr = dot(W_r[D:], H)
    cst = jnp.concatenate([dot(b_e, H), dot(b_r, H)]).reshape(1, 2 * D)
    A = dot(G, in_w) * 0.5
    v = dot(loop_rel, H)[0]
    va, vb = v[:HF], v[HF:]
    m = jnp.asarray(_M_NP)
    Dv = jnp.concatenate(
        [jnp.concatenate([jnp.diag(va), jnp.diag(m * vb)], axis=1),
         jnp.concatenate([jnp.diag(m * vb), jnp.diag((1 - m) * vb - m * va)],
                         axis=1)], axis=0)
    Lc = dot(H, dot(Dv, dot(G, loop_w))) * 0.5
    PeL = jnp.concatenate([P_e, Lc], axis=1)
    PrW = jnp.concatenate([P_r, w_rel], axis=1)
    CeCr = jnp.concatenate([Ce, Cr], axis=1)

    f32 = jnp.float32
    BN = 2000
    fxa, xloop = pl.pallas_call(
        _prep_body,
        grid=(N // BN,),
        in_specs=[pl.BlockSpec((BN, D), lambda i: (i, 0)),
                  pl.BlockSpec((D, 2 * D), lambda i: (0, 0))],
        out_specs=[pl.BlockSpec((BN, D), lambda i: (i, 0)),
                   pl.BlockSpec((BN, D), lambda i: (i, 0))],
        out_shape=[jax.ShapeDtypeStruct((N, D), f32),
                   jax.ShapeDtypeStruct((N, D), f32)],
    )(x, PeL)
    frb, rel_out = pl.pallas_call(
        _rel_body,
        out_shape=[jax.ShapeDtypeStruct((R, D), f32),
                   jax.ShapeDtypeStruct((R, D), f32)],
    )(rel_repr, PrW)

    e_per_w = E // NW
    mesh = plsc.VectorSubcoreMesh(core_axis_name="c", subcore_axis_name="s")
    gather = functools.partial(
        pl.kernel,
        mesh=mesh,
        out_type=[jax.ShapeDtypeStruct((E, D), f32),
                  jax.ShapeDtypeStruct((E, D), f32)],
        scratch_types=[pltpu.VMEM((e_per_w,), jnp.int32),
                       pltpu.VMEM((e_per_w,), jnp.int32)]
                      + [pltpu.VMEM((GCH, D), f32)] * (2 * NB)
                      + [pltpu.SemaphoreType.DMA] * (4 * NB),
    )(functools.partial(_gather_body, e_per_w=e_per_w))
    ga, gb = gather(fxa, frb, src, edge_type)

    BE = 2000
    grid = E // BE
    cecr_hi = CeCr.astype(jnp.bfloat16)
    cecr_lo = (CeCr - cecr_hi.astype(f32)).astype(jnp.bfloat16)
    P = pl.pallas_call(
        _edge_body,
        grid=(grid,),
        in_specs=[
            pl.BlockSpec((BE, D), lambda i: (i, 0)),
            pl.BlockSpec((BE, D), lambda i: (i, 0)),
            pl.BlockSpec((BE, D), lambda i: (i, 0)),
            pl.BlockSpec((BE, 1), lambda i: (i, 0)),
            pl.BlockSpec((D, 2 * D), lambda i: (0, 0)),
            pl.BlockSpec((D, 2 * D), lambda i: (0, 0)),
            pl.BlockSpec((1, 2 * D), lambda i: (0, 0)),
        ],
        out_specs=pl.BlockSpec((BE, D), lambda i: (i, 0)),
        out_shape=jax.ShapeDtypeStruct((E, D), f32),
    )(time_emd, ga, gb, edge_norm.reshape(E, 1), cecr_hi, cecr_lo, cst)

    e_per_s = E // NS
    npad = ((N + 8 * NS - 1) // (8 * NS)) * (8 * NS)
    zero = jnp.zeros((npad // NS, HF), f32)
    scatter = functools.partial(
        pl.kernel,
        mesh=plsc.VectorSubcoreMesh(core_axis_name="c", subcore_axis_name="s"),
        out_type=jax.ShapeDtypeStruct((npad, D), f32),
        scratch_types=[pltpu.VMEM((SCH,), jnp.int32)] * NB
                      + [pltpu.VMEM((SCH, HF), f32)] * NB
                      + [pltpu.SemaphoreType.DMA] * (2 * NB)
                      + [pltpu.VMEM_SHARED((npad, HF), f32)],
    )(functools.partial(_scatter_body, e_per_s=e_per_s, n_rows=npad))
    hsum = scatter(P, dst, zero)

    pre = pl.pallas_call(
        _post_body,
        grid=(N // BN,),
        in_specs=[pl.BlockSpec((BN, D), lambda i: (i, 0)),
                  pl.BlockSpec((BN, D), lambda i: (i, 0)),
                  pl.BlockSpec((D, D), lambda i: (0, 0))],
        out_specs=pl.BlockSpec((BN, D), lambda i: (i, 0)),
        out_shape=jax.ShapeDtypeStruct((N, D), f32),
    )(hsum, xloop, A)
    BC = 128
    out = pl.pallas_call(
        _bn_body,
        grid=(D // BC,),
        in_specs=[pl.BlockSpec((N, BC), lambda j: (0, j)),
                  pl.BlockSpec((1, BC), lambda j: (0, j)),
                  pl.BlockSpec((1, BC), lambda j: (0, j))],
        out_specs=pl.BlockSpec((N, BC), lambda j: (0, j)),
        out_shape=jax.ShapeDtypeStruct((N, D), f32),
    )(pre, gamma.reshape(1, D), beta.reshape(1, D))

    return out, rel_out


# speedup vs baseline: 10.3149x; 1.1915x over previous
import functools

import numpy as np
import jax
import jax.numpy as jnp
from jax import lax
from jax.experimental import pallas as pl
from jax.experimental.pallas import tpu as pltpu
from jax.experimental.pallas import tpu_sc as plsc

D = 256
HF = 128
_HIGH = jax.lax.Precision.HIGHEST

_Feye = np.fft.rfft(np.eye(D), axis=-1)
_H_NP = np.concatenate([_Feye.real, _Feye.imag[:, 1:HF]], axis=1).astype(np.float32)


def _unpack_np(p):
    re = p[..., 0:HF + 1]
    z = np.zeros(p.shape[:-1] + (1,))
    im = np.concatenate([z, p[..., HF + 1:D], z], axis=-1)
    return re + 1j * im


_G_NP = np.fft.irfft(_unpack_np(np.eye(D)), n=D, axis=-1).astype(np.float32)
_M_NP = (np.arange(HF) != 0).astype(np.float32)

NC = 2
NS = 16
NW = NC * NS


def _pack_bf16_pair(a, b):
    ua = jax.lax.bitcast_convert_type(a, jnp.uint32)
    ub = jax.lax.bitcast_convert_type(b, jnp.uint32)
    ra = (ua + jnp.uint32(0x7FFF) + ((ua >> 16) & jnp.uint32(1)))
    rb = (ub + jnp.uint32(0x7FFF) + ((ub >> 16) & jnp.uint32(1)))
    w = (ra & jnp.uint32(0xFFFF0000)) | (rb >> 16)
    return jax.lax.bitcast_convert_type(w, jnp.int32)


def _unpack_bf16_pair(w):
    uw = jax.lax.bitcast_convert_type(w, jnp.uint32)
    a = jax.lax.bitcast_convert_type(uw & jnp.uint32(0xFFFF0000), jnp.float32)
    b = jax.lax.bitcast_convert_type(uw << 16, jnp.float32)
    return a, b


def _prep_body(x_ref, pel_ref, fxa_ref, xloop_ref):
    xw = jnp.dot(x_ref[...], pel_ref[...], precision=_HIGH,
                 preferred_element_type=jnp.float32)
    fxa_ref[...] = _pack_bf16_pair(xw[:, :HF], xw[:, HF:D])
    xloop_ref[...] = xw[:, D:]


def _rel_body(rel_ref, prw_ref, frb_ref, relout_ref):
    rw = jnp.dot(rel_ref[...], prw_ref[...], precision=_HIGH,
                 preferred_element_type=jnp.float32)
    frb_ref[...] = _pack_bf16_pair(rw[:, :HF], rw[:, HF:D])
    relout_ref[...] = rw[:, D:]


GCH = 40


NB = 5


def _gather_body(fxa_hbm, frb_hbm, src_hbm, et_hbm, ga_hbm, gb_hbm,
                 src_v, et_v, *bufs, e_per_w):
    ra = bufs[0:NB]
    rb = bufs[NB:2 * NB]
    gsa = bufs[2 * NB:3 * NB]
    gsb = bufs[3 * NB:4 * NB]
    wsa = bufs[4 * NB:5 * NB]
    wsb = bufs[5 * NB:6 * NB]
    wid = lax.axis_index("s") * NC + lax.axis_index("c")
    base = wid * e_per_w
    nch = e_per_w // GCH
    pltpu.sync_copy(src_hbm.at[pl.ds(base, e_per_w)], src_v)
    pltpu.sync_copy(et_hbm.at[pl.ds(base, e_per_w)], et_v)

    def issue_gather(c, b):
        sl = pl.ds(c * GCH, GCH)
        pltpu.async_copy(fxa_hbm.at[src_v.at[sl]], ra[b], gsa[b])
        pltpu.async_copy(frb_hbm.at[et_v.at[sl]], rb[b], gsb[b])

    def wait_gather(c, b):
        sl = pl.ds(c * GCH, GCH)
        pltpu.make_async_copy(fxa_hbm.at[src_v.at[sl]], ra[b], gsa[b]).wait()
        pltpu.make_async_copy(frb_hbm.at[et_v.at[sl]], rb[b], gsb[b]).wait()

    def issue_write(c, b):
        sl = pl.ds(base + c * GCH, GCH)
        pltpu.async_copy(ra[b], ga_hbm.at[sl], wsa[b])
        pltpu.async_copy(rb[b], gb_hbm.at[sl], wsb[b])

    def drain_write(b):
        sl = pl.ds(base, GCH)
        pltpu.make_async_copy(ra[b], ga_hbm.at[sl], wsa[b]).wait()
        pltpu.make_async_copy(rb[b], gb_hbm.at[sl], wsb[b]).wait()

    for b in range(NB):
        issue_gather(b, b)

    def group(g, carry):
        c0 = g * NB
        for b in range(NB):
            wait_gather(c0 + b, b)
            issue_write(c0 + b, b)
        for b in range(NB):
            drain_write(b)

            @pl.when(c0 + NB + b < nch)
            def _():
                issue_gather(c0 + NB + b, b)

        return carry

    lax.fori_loop(0, nch // NB, group, 0)


def _edge_body(time_ref, ga_ref, gb_ref, norm_ref, whi_ref, wlo_ref,
               cst_ref, out_ref):
    th = time_ref[...].astype(jnp.bfloat16)
    dot16 = functools.partial(jnp.dot, preferred_element_type=jnp.float32)
    tf = dot16(th, whi_ref[...]) + dot16(th, wlo_ref[...])
    cst = cst_ref[...]
    gal, gah = _unpack_bf16_pair(ga_ref[...])
    gbl, gbh = _unpack_bf16_pair(gb_ref[...])
    ul = gal + tf[:, 0:HF] + cst[:, 0:HF]
    uh = gah + tf[:, HF:D] + cst[:, HF:D]
    vl = gbl + tf[:, D:D + HF] + cst[:, D:D + HF]
    vh = gbh + tf[:, D + HF:] + cst[:, D + HF:]
    s_lo = ul * vl
    s_hi = uh * vh
    lane = lax.broadcasted_iota(jnp.int32, (1, HF), 1)
    m = lane != 0
    p_lo = s_lo + jnp.where(m, s_hi, 0.0)
    p_hi = jnp.where(m, ul * vh - uh * vl, s_hi)
    out_ref[...] = jnp.concatenate([p_lo, p_hi], axis=1) * norm_ref[...]


SCH = 40


def _scatter_body(p_hbm, dst_hbm, zero_hbm, out_hbm, *bufs,
                  e_per_s, n_rows):
    dv = bufs[0:NB]
    rv = bufs[NB:2 * NB]
    dsem = bufs[2 * NB:3 * NB]
    psem = bufs[3 * NB:4 * NB]
    shared = bufs[4 * NB]
    c = lax.axis_index("c")
    s = lax.axis_index("s")
    rows_per_s = n_rows // NS
    nch = e_per_s // SCH
    pltpu.sync_copy(zero_hbm, shared.at[pl.ds(s * rows_per_s, rows_per_s)])
    plsc.subcore_barrier()

    def issue_load(g, b):
        e0 = s * e_per_s + g * SCH
        pltpu.async_copy(dst_hbm.at[pl.ds(e0, SCH)], dv[b], dsem[b])
        pltpu.async_copy(p_hbm.at[pl.ds(e0, SCH), pl.ds(c * HF, HF)],
                         rv[b], psem[b])

    def do_chunk(g, b):
        e0 = s * e_per_s + g * SCH
        pltpu.make_async_copy(dst_hbm.at[pl.ds(e0, SCH)], dv[b],
                              dsem[b]).wait()
        pltpu.make_async_copy(p_hbm.at[pl.ds(e0, SCH), pl.ds(c * HF, HF)],
                              rv[b], psem[b]).wait()
        pltpu.sync_copy(rv[b], shared.at[dv[b]], add=True)

    for b in range(NB):
        issue_load(b, b)

    def group(g, carry):
        c0 = g * NB
        for b in range(NB):
            do_chunk(c0 + b, b)

            @pl.when(c0 + NB + b < nch)
            def _():
                issue_load(c0 + NB + b, b)

        return carry

    lax.fori_loop(0, nch // NB, group, 0)
    plsc.subcore_barrier()
    pltpu.sync_copy(shared.at[pl.ds(s * rows_per_s, rows_per_s)],
                    out_hbm.at[pl.ds(s * rows_per_s, rows_per_s),
                               pl.ds(c * HF, HF)])


def _post_body(h_ref, xloop_ref, a_ref, pre_ref):
    pre_ref[...] = jnp.dot(h_ref[...], a_ref[...], precision=_HIGH,
                           preferred_element_type=jnp.float32) + xloop_ref[...]


def _bn_body(pre_ref, gamma_ref, beta_ref, out_ref):
    pre = pre_ref[...]
    n = pre.shape[0]
    mean = jnp.sum(pre, axis=0, keepdims=True) / n
    cen = pre - mean
    var = jnp.sum(cen * cen, axis=0, keepdims=True) / n
    out = cen * lax.rsqrt(var + 1e-5) * gamma_ref[...] + beta_ref[...]
    out_ref[...] = jnp.maximum(out, 0.0)


def kernel(x, rel_repr, edge_index, edge_type, edge_norm, time_emd,
           in_w, loop_w, w_rel, loop_rel, W_e, b_e, W_r, b_r, gamma, beta):
    N, _ = x.shape
    R, _ = rel_repr.shape
    E = edge_type.shape[0]
    src = edge_index[0]
    dst = edge_index[1]

    H = jnp.asarray(_H_NP)
    G = jnp.asarray(_G_NP)
    dot = functools.partial(jnp.dot, precision=_HIGH,
                            preferred_element_type=jnp.float32)
    P_e = dot(W_e[:D], H)
    Ce = dot(W_e[D:], H)
    P_r = dot(W_r[:D], H)
    Cr = dot(W_r[D:], H)
    cst = jnp.concatenate([dot(b_e, H), dot(b_r, H)]).reshape(1, 2 * D)
    A = dot(G, in_w) * 0.5
    v = dot(loop_rel, H)[0]
    va, vb = v[:HF], v[HF:]
    m = jnp.asarray(_M_NP)
    Dv = jnp.concatenate(
        [jnp.concatenate([jnp.diag(va), jnp.diag(m * vb)], axis=1),
         jnp.concatenate([jnp.diag(m * vb), jnp.diag((1 - m) * vb - m * va)],
                         axis=1)], axis=0)
    Lc = dot(H, dot(Dv, dot(G, loop_w))) * 0.5
    PeL = jnp.concatenate([P_e, Lc], axis=1)
    PrW = jnp.concatenate([P_r, w_rel], axis=1)
    CeCr = jnp.concatenate([Ce, Cr], axis=1)

    f32 = jnp.float32
    BN = 2000
    fxa, xloop = pl.pallas_call(
        _prep_body,
        grid=(N // BN,),
        in_specs=[pl.BlockSpec((BN, D), lambda i: (i, 0)),
                  pl.BlockSpec((D, 2 * D), lambda i: (0, 0))],
        out_specs=[pl.BlockSpec((BN, HF), lambda i: (i, 0)),
                   pl.BlockSpec((BN, D), lambda i: (i, 0))],
        out_shape=[jax.ShapeDtypeStruct((N, HF), jnp.int32),
                   jax.ShapeDtypeStruct((N, D), f32)],
    )(x, PeL)
    frb, rel_out = pl.pallas_call(
        _rel_body,
        out_shape=[jax.ShapeDtypeStruct((R, HF), jnp.int32),
                   jax.ShapeDtypeStruct((R, D), f32)],
    )(rel_repr, PrW)

    e_per_w = E // NW
    mesh = plsc.VectorSubcoreMesh(core_axis_name="c", subcore_axis_name="s")
    gather = functools.partial(
        pl.kernel,
        mesh=mesh,
        out_type=[jax.ShapeDtypeStruct((E, HF), jnp.int32),
                  jax.ShapeDtypeStruct((E, HF), jnp.int32)],
        scratch_types=[pltpu.VMEM((e_per_w,), jnp.int32),
                       pltpu.VMEM((e_per_w,), jnp.int32)]
                      + [pltpu.VMEM((GCH, HF), jnp.int32)] * (2 * NB)
                      + [pltpu.SemaphoreType.DMA] * (4 * NB),
    )(functools.partial(_gather_body, e_per_w=e_per_w))
    ga, gb = gather(fxa, frb, src, edge_type)

    BE = 2000
    grid = E // BE
    cecr_hi = CeCr.astype(jnp.bfloat16)
    cecr_lo = (CeCr - cecr_hi.astype(f32)).astype(jnp.bfloat16)
    P = pl.pallas_call(
        _edge_body,
        grid=(grid,),
        in_specs=[
            pl.BlockSpec((BE, D), lambda i: (i, 0)),
            pl.BlockSpec((BE, HF), lambda i: (i, 0)),
            pl.BlockSpec((BE, HF), lambda i: (i, 0)),
            pl.BlockSpec((BE, 1), lambda i: (i, 0)),
            pl.BlockSpec((D, 2 * D), lambda i: (0, 0)),
            pl.BlockSpec((D, 2 * D), lambda i: (0, 0)),
            pl.BlockSpec((1, 2 * D), lambda i: (0, 0)),
        ],
        out_specs=pl.BlockSpec((BE, D), lambda i: (i, 0)),
        out_shape=jax.ShapeDtypeStruct((E, D), f32),
    )(time_emd, ga, gb, edge_norm.reshape(E, 1), cecr_hi, cecr_lo, cst)

    e_per_s = E // NS
    npad = ((N + 8 * NS - 1) // (8 * NS)) * (8 * NS)
    zero = jnp.zeros((npad // NS, HF), f32)
    scatter = functools.partial(
        pl.kernel,
        mesh=plsc.VectorSubcoreMesh(core_axis_name="c", subcore_axis_name="s"),
        out_type=jax.ShapeDtypeStruct((npad, D), f32),
        scratch_types=[pltpu.VMEM((SCH,), jnp.int32)] * NB
                      + [pltpu.VMEM((SCH, HF), f32)] * NB
                      + [pltpu.SemaphoreType.DMA] * (2 * NB)
                      + [pltpu.VMEM_SHARED((npad, HF), f32)],
    )(functools.partial(_scatter_body, e_per_s=e_per_s, n_rows=npad))
    hsum = scatter(P, dst, zero)

    pre = pl.pallas_call(
        _post_body,
        grid=(N // BN,),
        in_specs=[pl.BlockSpec((BN, D), lambda i: (i, 0)),
                  pl.BlockSpec((BN, D), lambda i: (i, 0)),
                  pl.BlockSpec((D, D), lambda i: (0, 0))],
        out_specs=pl.BlockSpec((BN, D), lambda i: (i, 0)),
        out_shape=jax.ShapeDtypeStruct((N, D), f32),
    )(hsum, xloop, A)
    BC = 128
    out = pl.pallas_call(
        _bn_body,
        grid=(D // BC,),
        in_specs=[pl.BlockSpec((N, BC), lambda j: (0, j)),
                  pl.BlockSpec((1, BC), lambda j: (0, j)),
                  pl.BlockSpec((1, BC), lambda j: (0, j))],
        out_specs=pl.BlockSpec((N, BC), lambda j: (0, j)),
        out_shape=jax.ShapeDtypeStruct((N, D), f32),
    )(pre, gamma.reshape(1, D), beta.reshape(1, D))

    return out, rel_out


# speedup vs baseline: 10.7240x; 1.0397x over previous
import functools

import numpy as np
import jax
import jax.numpy as jnp
from jax import lax
from jax.experimental import pallas as pl
from jax.experimental.pallas import tpu as pltpu
from jax.experimental.pallas import tpu_sc as plsc

D = 256
HF = 128
_HIGH = jax.lax.Precision.HIGHEST

_Feye = np.fft.rfft(np.eye(D), axis=-1)
_H_NP = np.concatenate([_Feye.real, _Feye.imag[:, 1:HF]], axis=1).astype(np.float32)


def _unpack_np(p):
    re = p[..., 0:HF + 1]
    z = np.zeros(p.shape[:-1] + (1,))
    im = np.concatenate([z, p[..., HF + 1:D], z], axis=-1)
    return re + 1j * im


_G_NP = np.fft.irfft(_unpack_np(np.eye(D)), n=D, axis=-1).astype(np.float32)
_M_NP = (np.arange(HF) != 0).astype(np.float32)

NC = 2
NS = 16
NW = NC * NS


def _pack_bf16_pair(a, b):
    ua = jax.lax.bitcast_convert_type(a, jnp.uint32)
    ub = jax.lax.bitcast_convert_type(b, jnp.uint32)
    ra = (ua + jnp.uint32(0x7FFF) + ((ua >> 16) & jnp.uint32(1)))
    rb = (ub + jnp.uint32(0x7FFF) + ((ub >> 16) & jnp.uint32(1)))
    w = (ra & jnp.uint32(0xFFFF0000)) | (rb >> 16)
    return jax.lax.bitcast_convert_type(w, jnp.int32)


def _unpack_bf16_pair(w):
    uw = jax.lax.bitcast_convert_type(w, jnp.uint32)
    a = jax.lax.bitcast_convert_type(uw & jnp.uint32(0xFFFF0000), jnp.float32)
    b = jax.lax.bitcast_convert_type(uw << 16, jnp.float32)
    return a, b


def _prep_body(x_ref, pel_ref, fxa_ref, xloop_ref):
    xw = jnp.dot(x_ref[...], pel_ref[...], precision=_HIGH,
                 preferred_element_type=jnp.float32)
    fxa_ref[...] = _pack_bf16_pair(xw[:, :HF], xw[:, HF:D])
    xloop_ref[...] = xw[:, D:]


def _rel_body(rel_ref, prw_ref, frb_ref, relout_ref):
    rw = jnp.dot(rel_ref[...], prw_ref[...], precision=_HIGH,
                 preferred_element_type=jnp.float32)
    frb_ref[...] = _pack_bf16_pair(rw[:, :HF], rw[:, HF:D])
    relout_ref[...] = rw[:, D:]


GCH = 40


NB = 5


def _gather_body(fxa_hbm, frb_hbm, src_hbm, et_hbm, ga_hbm, gb_hbm,
                 src_v, et_v, *bufs, e_per_w, eoff):
    ra = bufs[0:NB]
    rb = bufs[NB:2 * NB]
    gsa = bufs[2 * NB:3 * NB]
    gsb = bufs[3 * NB:4 * NB]
    wsa = bufs[4 * NB:5 * NB]
    wsb = bufs[5 * NB:6 * NB]
    wid = lax.axis_index("s") * NC + lax.axis_index("c")
    base = wid * e_per_w
    nch = e_per_w // GCH
    pltpu.sync_copy(src_hbm.at[pl.ds(eoff + base, e_per_w)], src_v)
    pltpu.sync_copy(et_hbm.at[pl.ds(eoff + base, e_per_w)], et_v)

    def issue_gather(c, b):
        sl = pl.ds(c * GCH, GCH)
        pltpu.async_copy(fxa_hbm.at[src_v.at[sl]], ra[b], gsa[b])
        pltpu.async_copy(frb_hbm.at[et_v.at[sl]], rb[b], gsb[b])

    def wait_gather(c, b):
        sl = pl.ds(c * GCH, GCH)
        pltpu.make_async_copy(fxa_hbm.at[src_v.at[sl]], ra[b], gsa[b]).wait()
        pltpu.make_async_copy(frb_hbm.at[et_v.at[sl]], rb[b], gsb[b]).wait()

    def issue_write(c, b):
        sl = pl.ds(base + c * GCH, GCH)
        pltpu.async_copy(ra[b], ga_hbm.at[sl], wsa[b])
        pltpu.async_copy(rb[b], gb_hbm.at[sl], wsb[b])

    def drain_write(b):
        sl = pl.ds(base, GCH)
        pltpu.make_async_copy(ra[b], ga_hbm.at[sl], wsa[b]).wait()
        pltpu.make_async_copy(rb[b], gb_hbm.at[sl], wsb[b]).wait()

    for b in range(NB):
        issue_gather(b, b)

    def group(g, carry):
        c0 = g * NB
        for b in range(NB):
            wait_gather(c0 + b, b)
            issue_write(c0 + b, b)
        for b in range(NB):
            drain_write(b)

            @pl.when(c0 + NB + b < nch)
            def _():
                issue_gather(c0 + NB + b, b)

        return carry

    lax.fori_loop(0, nch // NB, group, 0)


def _edge_body(time_ref, ga_ref, gb_ref, norm_ref, whi_ref, wlo_ref,
               cst_ref, out_ref):
    th = time_ref[...].astype(jnp.bfloat16)
    dot16 = functools.partial(jnp.dot, preferred_element_type=jnp.float32)
    tf = dot16(th, whi_ref[...]) + dot16(th, wlo_ref[...])
    cst = cst_ref[...]
    gal, gah = _unpack_bf16_pair(ga_ref[...])
    gbl, gbh = _unpack_bf16_pair(gb_ref[...])
    ul = gal + tf[:, 0:HF] + cst[:, 0:HF]
    uh = gah + tf[:, HF:D] + cst[:, HF:D]
    vl = gbl + tf[:, D:D + HF] + cst[:, D:D + HF]
    vh = gbh + tf[:, D + HF:] + cst[:, D + HF:]
    s_lo = ul * vl
    s_hi = uh * vh
    lane = lax.broadcasted_iota(jnp.int32, (1, HF), 1)
    m = lane != 0
    p_lo = s_lo + jnp.where(m, s_hi, 0.0)
    p_hi = jnp.where(m, ul * vh - uh * vl, s_hi)
    out_ref[...] = jnp.concatenate([p_lo, p_hi], axis=1) * norm_ref[...]


SCH = 40


def _scatter_body(p_hbm, dst_hbm, zero_hbm, out_hbm, *bufs,
                  e_per_s, n_rows, eoff):
    dv = bufs[0:NB]
    rv = bufs[NB:2 * NB]
    dsem = bufs[2 * NB:3 * NB]
    psem = bufs[3 * NB:4 * NB]
    shared = bufs[4 * NB]
    c = lax.axis_index("c")
    s = lax.axis_index("s")
    rows_per_s = n_rows // NS
    nch = e_per_s // SCH
    pltpu.sync_copy(zero_hbm, shared.at[pl.ds(s * rows_per_s, rows_per_s)])
    plsc.subcore_barrier()

    def issue_load(g, b):
        e0 = s * e_per_s + g * SCH
        pltpu.async_copy(dst_hbm.at[pl.ds(eoff + e0, SCH)], dv[b], dsem[b])
        pltpu.async_copy(p_hbm.at[pl.ds(e0, SCH), pl.ds(c * HF, HF)],
                         rv[b], psem[b])

    def do_chunk(g, b):
        e0 = s * e_per_s + g * SCH
        pltpu.make_async_copy(dst_hbm.at[pl.ds(eoff + e0, SCH)], dv[b],
                              dsem[b]).wait()
        pltpu.make_async_copy(p_hbm.at[pl.ds(e0, SCH), pl.ds(c * HF, HF)],
                              rv[b], psem[b]).wait()
        pltpu.sync_copy(rv[b], shared.at[dv[b]], add=True)

    for b in range(NB):
        issue_load(b, b)

    def group(g, carry):
        c0 = g * NB
        for b in range(NB):
            do_chunk(c0 + b, b)

            @pl.when(c0 + NB + b < nch)
            def _():
                issue_load(c0 + NB + b, b)

        return carry

    lax.fori_loop(0, nch // NB, group, 0)
    plsc.subcore_barrier()
    pltpu.sync_copy(shared.at[pl.ds(s * rows_per_s, rows_per_s)],
                    out_hbm.at[pl.ds(s * rows_per_s, rows_per_s),
                               pl.ds(c * HF, HF)])


def _post_body(h1_ref, h2_ref, xloop_ref, a_ref, pre_ref):
    h = h1_ref[...] + h2_ref[...]
    pre_ref[...] = jnp.dot(h, a_ref[...], precision=_HIGH,
                           preferred_element_type=jnp.float32) + xloop_ref[...]


def _bn_body(pre_ref, gamma_ref, beta_ref, out_ref):
    pre = pre_ref[...]
    n = pre.shape[0]
    mean = jnp.sum(pre, axis=0, keepdims=True) / n
    cen = pre - mean
    var = jnp.sum(cen * cen, axis=0, keepdims=True) / n
    out = cen * lax.rsqrt(var + 1e-5) * gamma_ref[...] + beta_ref[...]
    out_ref[...] = jnp.maximum(out, 0.0)


def kernel(x, rel_repr, edge_index, edge_type, edge_norm, time_emd,
           in_w, loop_w, w_rel, loop_rel, W_e, b_e, W_r, b_r, gamma, beta):
    N, _ = x.shape
    R, _ = rel_repr.shape
    E = edge_type.shape[0]
    src = edge_index[0]
    dst = edge_index[1]

    H = jnp.asarray(_H_NP)
    G = jnp.asarray(_G_NP)
    dot = functools.partial(jnp.dot, precision=_HIGH,
                            preferred_element_type=jnp.float32)
    P_e = dot(W_e[:D], H)
    Ce = dot(W_e[D:], H)
    P_r = dot(W_r[:D], H)
    Cr = dot(W_r[D:], H)
    cst = jnp.concatenate([dot(b_e, H), dot(b_r, H)]).reshape(1, 2 * D)
    A = dot(G, in_w) * 0.5
    v = dot(loop_rel, H)[0]
    va, vb = v[:HF], v[HF:]
    m = jnp.asarray(_M_NP)
    Dv = jnp.concatenate(
        [jnp.concatenate([jnp.diag(va), jnp.diag(m * vb)], axis=1),
         jnp.concatenate([jnp.diag(m * vb), jnp.diag((1 - m) * vb - m * va)],
                         axis=1)], axis=0)
    Lc = dot(H, dot(Dv, dot(G, loop_w))) * 0.5
    PeL = jnp.concatenate([P_e, Lc], axis=1)
    PrW = jnp.concatenate([P_r, w_rel], axis=1)
    CeCr = jnp.concatenate([Ce, Cr], axis=1)

    f32 = jnp.float32
    BN = 2000
    fxa, xloop = pl.pallas_call(
        _prep_body,
        grid=(N // BN,),
        in_specs=[pl.BlockSpec((BN, D), lambda i: (i, 0)),
                  pl.BlockSpec((D, 2 * D), lambda i: (0, 0))],
        out_specs=[pl.BlockSpec((BN, HF), lambda i: (i, 0)),
                   pl.BlockSpec((BN, D), lambda i: (i, 0))],
        out_shape=[jax.ShapeDtypeStruct((N, HF), jnp.int32),
                   jax.ShapeDtypeStruct((N, D), f32)],
    )(x, PeL)
    frb, rel_out = pl.pallas_call(
        _rel_body,
        out_shape=[jax.ShapeDtypeStruct((R, HF), jnp.int32),
                   jax.ShapeDtypeStruct((R, D), f32)],
    )(rel_repr, PrW)

    E1 = 83200
    halves = [(0, E1), (E1, E - E1)]
    BE = 1600
    npad = ((N + 8 * NS - 1) // (8 * NS)) * (8 * NS)
    zero = jnp.zeros((npad // NS, HF), f32)
    cecr_hi = CeCr.astype(jnp.bfloat16)
    cecr_lo = (CeCr - cecr_hi.astype(f32)).astype(jnp.bfloat16)
    norm2d = edge_norm.reshape(E, 1)
    mesh = plsc.VectorSubcoreMesh(core_axis_name="c", subcore_axis_name="s")

    gas, gbs, hs = [], [], []
    for eoff, esz in halves:
        e_per_w = esz // NW
        gather = functools.partial(
            pl.kernel,
            mesh=mesh,
            out_type=[jax.ShapeDtypeStruct((esz, HF), jnp.int32),
                      jax.ShapeDtypeStruct((esz, HF), jnp.int32)],
            scratch_types=[pltpu.VMEM((e_per_w,), jnp.int32),
                           pltpu.VMEM((e_per_w,), jnp.int32)]
                          + [pltpu.VMEM((GCH, HF), jnp.int32)] * (2 * NB)
                          + [pltpu.SemaphoreType.DMA] * (4 * NB),
        )(functools.partial(_gather_body, e_per_w=e_per_w, eoff=eoff))
        ga, gb = gather(fxa, frb, src, edge_type)
        gas.append(ga)
        gbs.append(gb)

    ps = []
    for k, (eoff, esz) in enumerate(halves):
        offb = eoff // BE
        P = pl.pallas_call(
            _edge_body,
            grid=(esz // BE,),
            in_specs=[
                pl.BlockSpec((BE, D), lambda i, o=offb: (i + o, 0)),
                pl.BlockSpec((BE, HF), lambda i: (i, 0)),
                pl.BlockSpec((BE, HF), lambda i: (i, 0)),
                pl.BlockSpec((BE, 1), lambda i, o=offb: (i + o, 0)),
                pl.BlockSpec((D, 2 * D), lambda i: (0, 0)),
                pl.BlockSpec((D, 2 * D), lambda i: (0, 0)),
                pl.BlockSpec((1, 2 * D), lambda i: (0, 0)),
            ],
            out_specs=pl.BlockSpec((BE, D), lambda i: (i, 0)),
            out_shape=jax.ShapeDtypeStruct((esz, D), f32),
        )(time_emd, gas[k], gbs[k], norm2d, cecr_hi, cecr_lo, cst)
        ps.append(P)

    for k, (eoff, esz) in enumerate(halves):
        e_per_s = esz // NS
        scatter = functools.partial(
            pl.kernel,
            mesh=mesh,
            out_type=jax.ShapeDtypeStruct((npad, D), f32),
            scratch_types=[pltpu.VMEM((SCH,), jnp.int32)] * NB
                          + [pltpu.VMEM((SCH, HF), f32)] * NB
                          + [pltpu.SemaphoreType.DMA] * (2 * NB)
                          + [pltpu.VMEM_SHARED((npad, HF), f32)],
        )(functools.partial(_scatter_body, e_per_s=e_per_s, n_rows=npad,
                            eoff=eoff))
        hs.append(scatter(ps[k], dst, zero))

    pre = pl.pallas_call(
        _post_body,
        grid=(N // BN,),
        in_specs=[pl.BlockSpec((BN, D), lambda i: (i, 0)),
                  pl.BlockSpec((BN, D), lambda i: (i, 0)),
                  pl.BlockSpec((BN, D), lambda i: (i, 0)),
                  pl.BlockSpec((D, D), lambda i: (0, 0))],
        out_specs=pl.BlockSpec((BN, D), lambda i: (i, 0)),
        out_shape=jax.ShapeDtypeStruct((N, D), f32),
    )(hs[0], hs[1], xloop, A)
    BC = 128
    out = pl.pallas_call(
        _bn_body,
        grid=(D // BC,),
        in_specs=[pl.BlockSpec((N, BC), lambda j: (0, j)),
                  pl.BlockSpec((1, BC), lambda j: (0, j)),
                  pl.BlockSpec((1, BC), lambda j: (0, j))],
        out_specs=pl.BlockSpec((N, BC), lambda j: (0, j)),
        out_shape=jax.ShapeDtypeStruct((N, D), f32),
    )(pre, gamma.reshape(1, D), beta.reshape(1, D))

    return out, rel_out


# speedup vs baseline: 10.9748x; 1.0234x over previous
import functools

import numpy as np
import jax
import jax.numpy as jnp
from jax import lax
from jax.experimental import pallas as pl
from jax.experimental.pallas import tpu as pltpu
from jax.experimental.pallas import tpu_sc as plsc

D = 256
HF = 128
_HIGH = jax.lax.Precision.HIGHEST

_Feye = np.fft.rfft(np.eye(D), axis=-1)
_H_NP = np.concatenate([_Feye.real, _Feye.imag[:, 1:HF]], axis=1).astype(np.float32)


def _unpack_np(p):
    re = p[..., 0:HF + 1]
    z = np.zeros(p.shape[:-1] + (1,))
    im = np.concatenate([z, p[..., HF + 1:D], z], axis=-1)
    return re + 1j * im


_G_NP = np.fft.irfft(_unpack_np(np.eye(D)), n=D, axis=-1).astype(np.float32)
_M_NP = (np.arange(HF) != 0).astype(np.float32)

NC = 2
NS = 16
NW = NC * NS


def _pack_bf16_pair(a, b):
    ua = jax.lax.bitcast_convert_type(a, jnp.uint32)
    ub = jax.lax.bitcast_convert_type(b, jnp.uint32)
    ra = (ua + jnp.uint32(0x7FFF) + ((ua >> 16) & jnp.uint32(1)))
    rb = (ub + jnp.uint32(0x7FFF) + ((ub >> 16) & jnp.uint32(1)))
    w = (ra & jnp.uint32(0xFFFF0000)) | (rb >> 16)
    return jax.lax.bitcast_convert_type(w, jnp.int32)


def _unpack_bf16_pair(w):
    uw = jax.lax.bitcast_convert_type(w, jnp.uint32)
    a = jax.lax.bitcast_convert_type(uw & jnp.uint32(0xFFFF0000), jnp.float32)
    b = jax.lax.bitcast_convert_type(uw << 16, jnp.float32)
    return a, b


def _prep_body(x_ref, pel_ref, fxa_ref, xloop_ref):
    xw = jnp.dot(x_ref[...], pel_ref[...], precision=_HIGH,
                 preferred_element_type=jnp.float32)
    fxa_ref[...] = _pack_bf16_pair(xw[:, :HF], xw[:, HF:D])
    xloop_ref[...] = xw[:, D:]


def _rel_body(rel_ref, prw_ref, frb_ref, relout_ref):
    rw = jnp.dot(rel_ref[...], prw_ref[...], precision=_HIGH,
                 preferred_element_type=jnp.float32)
    frb_ref[...] = _pack_bf16_pair(rw[:, :HF], rw[:, HF:D])
    relout_ref[...] = rw[:, D:]


GCH = 40


NB = 5


def _gather_body(fxa_hbm, frb_hbm, src_hbm, et_hbm, ga_hbm, gb_hbm,
                 src_v, et_v, *bufs, e_per_w, eoff):
    ra = bufs[0:NB]
    rb = bufs[NB:2 * NB]
    gsa = bufs[2 * NB:3 * NB]
    gsb = bufs[3 * NB:4 * NB]
    wsa = bufs[4 * NB:5 * NB]
    wsb = bufs[5 * NB:6 * NB]
    wid = lax.axis_index("s") * NC + lax.axis_index("c")
    base = wid * e_per_w
    nch = e_per_w // GCH
    pltpu.sync_copy(src_hbm.at[pl.ds(eoff + base, e_per_w)], src_v)
    pltpu.sync_copy(et_hbm.at[pl.ds(eoff + base, e_per_w)], et_v)

    def issue_gather(c, b):
        sl = pl.ds(c * GCH, GCH)
        pltpu.async_copy(fxa_hbm.at[src_v.at[sl]], ra[b], gsa[b])
        pltpu.async_copy(frb_hbm.at[et_v.at[sl]], rb[b], gsb[b])

    def wait_gather(c, b):
        sl = pl.ds(c * GCH, GCH)
        pltpu.make_async_copy(fxa_hbm.at[src_v.at[sl]], ra[b], gsa[b]).wait()
        pltpu.make_async_copy(frb_hbm.at[et_v.at[sl]], rb[b], gsb[b]).wait()

    def issue_write(c, b):
        sl = pl.ds(base + c * GCH, GCH)
        pltpu.async_copy(ra[b], ga_hbm.at[sl], wsa[b])
        pltpu.async_copy(rb[b], gb_hbm.at[sl], wsb[b])

    def drain_write(b):
        sl = pl.ds(base, GCH)
        pltpu.make_async_copy(ra[b], ga_hbm.at[sl], wsa[b]).wait()
        pltpu.make_async_copy(rb[b], gb_hbm.at[sl], wsb[b]).wait()

    for b in range(NB):
        issue_gather(b, b)

    def group(g, carry):
        c0 = g * NB
        for b in range(NB):
            wait_gather(c0 + b, b)
            issue_write(c0 + b, b)
        for b in range(NB):
            drain_write(b)

            @pl.when(c0 + NB + b < nch)
            def _():
                issue_gather(c0 + NB + b, b)

        return carry

    lax.fori_loop(0, nch // NB, group, 0)


def _edge_body(time_ref, ga_ref, gb_ref, norm_ref, whi_ref, wlo_ref,
               cst_ref, out_ref):
    th = time_ref[...].astype(jnp.bfloat16)
    dot16 = functools.partial(jnp.dot, preferred_element_type=jnp.float32)
    tf = dot16(th, whi_ref[...]) + dot16(th, wlo_ref[...])
    cst = cst_ref[...]
    gal, gah = _unpack_bf16_pair(ga_ref[...])
    gbl, gbh = _unpack_bf16_pair(gb_ref[...])
    ul = gal + tf[:, 0:HF] + cst[:, 0:HF]
    uh = gah + tf[:, HF:D] + cst[:, HF:D]
    vl = gbl + tf[:, D:D + HF] + cst[:, D:D + HF]
    vh = gbh + tf[:, D + HF:] + cst[:, D + HF:]
    s_lo = ul * vl
    s_hi = uh * vh
    lane = lax.broadcasted_iota(jnp.int32, (1, HF), 1)
    m = lane != 0
    p_lo = s_lo + jnp.where(m, s_hi, 0.0)
    p_hi = jnp.where(m, ul * vh - uh * vl, s_hi)
    out_ref[...] = jnp.concatenate([p_lo, p_hi], axis=1) * norm_ref[...]


SCH = 40


def _scatter_body(p_hbm, dst_hbm, zero_hbm, out_hbm, *bufs,
                  e_per_s, n_rows, eoff):
    dv = bufs[0:NB]
    rv = bufs[NB:2 * NB]
    dsem = bufs[2 * NB:3 * NB]
    psem = bufs[3 * NB:4 * NB]
    shared = bufs[4 * NB]
    c = lax.axis_index("c")
    s = lax.axis_index("s")
    rows_per_s = n_rows // NS
    nch = e_per_s // SCH
    pltpu.sync_copy(zero_hbm, shared.at[pl.ds(s * rows_per_s, rows_per_s)])
    plsc.subcore_barrier()

    def issue_load(g, b):
        e0 = s * e_per_s + g * SCH
        pltpu.async_copy(dst_hbm.at[pl.ds(eoff + e0, SCH)], dv[b], dsem[b])
        pltpu.async_copy(p_hbm.at[pl.ds(e0, SCH), pl.ds(c * HF, HF)],
                         rv[b], psem[b])

    def do_chunk(g, b):
        e0 = s * e_per_s + g * SCH
        pltpu.make_async_copy(dst_hbm.at[pl.ds(eoff + e0, SCH)], dv[b],
                              dsem[b]).wait()
        pltpu.make_async_copy(p_hbm.at[pl.ds(e0, SCH), pl.ds(c * HF, HF)],
                              rv[b], psem[b]).wait()
        pltpu.sync_copy(rv[b], shared.at[dv[b]], add=True)

    for b in range(NB):
        issue_load(b, b)

    def group(g, carry):
        c0 = g * NB
        for b in range(NB):
            do_chunk(c0 + b, b)

            @pl.when(c0 + NB + b < nch)
            def _():
                issue_load(c0 + NB + b, b)

        return carry

    lax.fori_loop(0, nch // NB, group, 0)
    plsc.subcore_barrier()
    pltpu.sync_copy(shared.at[pl.ds(s * rows_per_s, rows_per_s)],
                    out_hbm.at[pl.ds(s * rows_per_s, rows_per_s),
                               pl.ds(c * HF, HF)])


def _post_body(h1_ref, h2_ref, xloop_ref, a_ref, pre_ref):
    h = h1_ref[...] + h2_ref[...]
    pre_ref[...] = jnp.dot(h, a_ref[...], precision=_HIGH,
                           preferred_element_type=jnp.float32) + xloop_ref[...]


def _bn_body(pre_ref, gamma_ref, beta_ref, out_ref):
    pre = pre_ref[...]
    n = pre.shape[0]
    mean = jnp.sum(pre, axis=0, keepdims=True) / n
    cen = pre - mean
    var = jnp.sum(cen * cen, axis=0, keepdims=True) / n
    out = cen * lax.rsqrt(var + 1e-5) * gamma_ref[...] + beta_ref[...]
    out_ref[...] = jnp.maximum(out, 0.0)


def kernel(x, rel_repr, edge_index, edge_type, edge_norm, time_emd,
           in_w, loop_w, w_rel, loop_rel, W_e, b_e, W_r, b_r, gamma, beta):
    N, _ = x.shape
    R, _ = rel_repr.shape
    E = edge_type.shape[0]
    src = edge_index[0]
    dst = edge_index[1]

    H = jnp.asarray(_H_NP)
    G = jnp.asarray(_G_NP)
    dot = functools.partial(jnp.dot, precision=_HIGH,
                            preferred_element_type=jnp.float32)
    P_e = dot(W_e[:D], H)
    Ce = dot(W_e[D:], H)
    P_r = dot(W_r[:D], H)
    Cr = dot(W_r[D:], H)
    cst = jnp.concatenate([dot(b_e, H), dot(b_r, H)]).reshape(1, 2 * D)
    A = dot(G, in_w) * 0.5
    v = dot(loop_rel, H)[0]
    va, vb = v[:HF], v[HF:]
    m = jnp.asarray(_M_NP)
    Dv = jnp.concatenate(
        [jnp.concatenate([jnp.diag(va), jnp.diag(m * vb)], axis=1),
         jnp.concatenate([jnp.diag(m * vb), jnp.diag((1 - m) * vb - m * va)],
                         axis=1)], axis=0)
    Lc = dot(H, dot(Dv, dot(G, loop_w))) * 0.5
    PeL = jnp.concatenate([P_e, Lc], axis=1)
    PrW = jnp.concatenate([P_r, w_rel], axis=1)
    CeCr = jnp.concatenate([Ce, Cr], axis=1)

    f32 = jnp.float32
    BN = 2000
    fxa, xloop = pl.pallas_call(
        _prep_body,
        grid=(N // BN,),
        in_specs=[pl.BlockSpec((BN, D), lambda i: (i, 0)),
                  pl.BlockSpec((D, 2 * D), lambda i: (0, 0))],
        out_specs=[pl.BlockSpec((BN, HF), lambda i: (i, 0)),
                   pl.BlockSpec((BN, D), lambda i: (i, 0))],
        out_shape=[jax.ShapeDtypeStruct((N, HF), jnp.int32),
                   jax.ShapeDtypeStruct((N, D), f32)],
    )(x, PeL)
    frb, rel_out = pl.pallas_call(
        _rel_body,
        out_shape=[jax.ShapeDtypeStruct((R, HF), jnp.int32),
                   jax.ShapeDtypeStruct((R, D), f32)],
    )(rel_repr, PrW)

    E1 = 70400
    halves = [(0, E1), (E1, E - E1)]
    BE = 3200
    npad = ((N + 8 * NS - 1) // (8 * NS)) * (8 * NS)
    zero = jnp.zeros((npad // NS, HF), f32)
    cecr_hi = CeCr.astype(jnp.bfloat16)
    cecr_lo = (CeCr - cecr_hi.astype(f32)).astype(jnp.bfloat16)
    norm2d = edge_norm.reshape(E, 1)
    mesh = plsc.VectorSubcoreMesh(core_axis_name="c", subcore_axis_name="s")

    gas, gbs, hs = [], [], []
    for eoff, esz in halves:
        e_per_w = esz // NW
        gather = functools.partial(
            pl.kernel,
            mesh=mesh,
            out_type=[jax.ShapeDtypeStruct((esz, HF), jnp.int32),
                      jax.ShapeDtypeStruct((esz, HF), jnp.int32)],
            scratch_types=[pltpu.VMEM((e_per_w,), jnp.int32),
                           pltpu.VMEM((e_per_w,), jnp.int32)]
                          + [pltpu.VMEM((GCH, HF), jnp.int32)] * (2 * NB)
                          + [pltpu.SemaphoreType.DMA] * (4 * NB),
        )(functools.partial(_gather_body, e_per_w=e_per_w, eoff=eoff))
        ga, gb = gather(fxa, frb, src, edge_type)
        gas.append(ga)
        gbs.append(gb)

    ps = []
    for k, (eoff, esz) in enumerate(halves):
        offb = eoff // BE
        P = pl.pallas_call(
            _edge_body,
            grid=(esz // BE,),
            in_specs=[
                pl.BlockSpec((BE, D), lambda i, o=offb: (i + o, 0)),
                pl.BlockSpec((BE, HF), lambda i: (i, 0)),
                pl.BlockSpec((BE, HF), lambda i: (i, 0)),
                pl.BlockSpec((BE, 1), lambda i, o=offb: (i + o, 0)),
                pl.BlockSpec((D, 2 * D), lambda i: (0, 0)),
                pl.BlockSpec((D, 2 * D), lambda i: (0, 0)),
                pl.BlockSpec((1, 2 * D), lambda i: (0, 0)),
            ],
            out_specs=pl.BlockSpec((BE, D), lambda i: (i, 0)),
            out_shape=jax.ShapeDtypeStruct((esz, D), f32),
        )(time_emd, gas[k], gbs[k], norm2d, cecr_hi, cecr_lo, cst)
        ps.append(P)

    for k, (eoff, esz) in enumerate(halves):
        e_per_s = esz // NS
        scatter = functools.partial(
            pl.kernel,
            mesh=mesh,
            out_type=jax.ShapeDtypeStruct((npad, D), f32),
            scratch_types=[pltpu.VMEM((SCH,), jnp.int32)] * NB
                          + [pltpu.VMEM((SCH, HF), f32)] * NB
                          + [pltpu.SemaphoreType.DMA] * (2 * NB)
                          + [pltpu.VMEM_SHARED((npad, HF), f32)],
        )(functools.partial(_scatter_body, e_per_s=e_per_s, n_rows=npad,
                            eoff=eoff))
        hs.append(scatter(ps[k], dst, zero))

    pre = pl.pallas_call(
        _post_body,
        grid=(N // BN,),
        in_specs=[pl.BlockSpec((BN, D), lambda i: (i, 0)),
                  pl.BlockSpec((BN, D), lambda i: (i, 0)),
                  pl.BlockSpec((BN, D), lambda i: (i, 0)),
                  pl.BlockSpec((D, D), lambda i: (0, 0))],
        out_specs=pl.BlockSpec((BN, D), lambda i: (i, 0)),
        out_shape=jax.ShapeDtypeStruct((N, D), f32),
    )(hs[0], hs[1], xloop, A)
    BC = 128
    out = pl.pallas_call(
        _bn_body,
        grid=(D // BC,),
        in_specs=[pl.BlockSpec((N, BC), lambda j: (0, j)),
                  pl.BlockSpec((1, BC), lambda j: (0, j)),
                  pl.BlockSpec((1, BC), lambda j: (0, j))],
        out_specs=pl.BlockSpec((N, BC), lambda j: (0, j)),
        out_shape=jax.ShapeDtypeStruct((N, D), f32),
    )(pre, gamma.reshape(1, D), beta.reshape(1, D))

    return out, rel_out
